# Initial kernel scaffold; baseline (speedup 1.0000x reference)
import jax
import jax.numpy as jnp
from jax.experimental import pallas as pl


def kernel(x, edge_index, edge_attr, params):
    raise NotImplementedError("write your pallas kernel here")


# speedup vs baseline: 1.6068x; 1.6068x over previous
import math

import jax
import jax.numpy as jnp
from jax.experimental import pallas as pl
from jax.experimental.pallas import tpu as pltpu

G = 32
NP = 512
EP = NP * 16
N = G * NP
E = G * EP
DIN = 128
BSH = 256
NOUT = 10
ALPHA = 0.5
BETA = math.log(0.1 / 2.0 + 1.0)

_INTERPRET = False

_HI = jax.lax.Precision.HIGHEST


def _f32(*shape):
    return jax.ShapeDtypeStruct(shape, jnp.float32)


def _bn_relu(pre, psum, psumsq, gam, bet):
    mean = jnp.sum(psum[:, 0, :], axis=0) / N
    msq = jnp.sum(psumsq[:, 0, :], axis=0) / N
    var = msq - mean * mean
    inv = gam[0] / jnp.sqrt(var + 1e-5)
    return jnp.maximum((pre - mean[None, :]) * inv[None, :] + bet[0][None, :], 0.0)


def _colsums(o):
    return jnp.sum(o, axis=0)[None, :], jnp.sum(o * o, axis=0)[None, :]


def _gcn_body(first):
    if first:
        def body(pre_ref, a_ref, w_ref, b_ref, out_ref, ps_ref, pq_ref):
            xn = pre_ref[0]
            h = jnp.dot(xn, w_ref[...], preferred_element_type=jnp.float32)
            o = jnp.dot(a_ref[0], h, preferred_element_type=jnp.float32,
                        precision=_HI) + b_ref[0][None, :]
            out_ref[0] = o
            ps_ref[0], pq_ref[0] = _colsums(o)
        return body

    def body(pre_ref, ps_in, pq_in, g_ref, be_ref, a_ref, w_ref, b_ref,
             out_ref, ps_ref, pq_ref):
        xn = _bn_relu(pre_ref[0], ps_in[...], pq_in[...], g_ref[...], be_ref[...])
        h = jnp.dot(xn, w_ref[...], preferred_element_type=jnp.float32)
        o = jnp.dot(a_ref[0], h, preferred_element_type=jnp.float32,
                    precision=_HI) + b_ref[0][None, :]
        out_ref[0] = o
        ps_ref[0], pq_ref[0] = _colsums(o)
    return body


def _full(shape):
    nd = len(shape)
    return pl.BlockSpec(shape, lambda g: (0,) * nd)


def _gcn_layer(pre, stats, gambet, A, W, b, first=False):
    cin = pre.shape[-1]
    cout = W.shape[-1]
    in_specs = [pl.BlockSpec((1, NP, cin), lambda g: (g, 0, 0))]
    args = [pre]
    if not first:
        ps, pq = stats
        gam, bet = gambet
        in_specs += [_full((G, 1, cin)), _full((G, 1, cin)),
                     _full((1, cin)), _full((1, cin))]
        args += [ps, pq, gam, bet]
    in_specs += [pl.BlockSpec((1, NP, NP), lambda g: (g, 0, 0)),
                 _full(W.shape), _full((1, cout))]
    args += [A, W, b.reshape(1, cout)]
    out, ps2, pq2 = pl.pallas_call(
        _gcn_body(first),
        grid=(G,),
        in_specs=in_specs,
        out_specs=[pl.BlockSpec((1, NP, cout), lambda g: (g, 0, 0)),
                   pl.BlockSpec((1, 1, cout), lambda g: (g, 0, 0)),
                   pl.BlockSpec((1, 1, cout), lambda g: (g, 0, 0))],
        out_shape=[_f32(G, NP, cout), _f32(G, 1, cout), _f32(G, 1, cout)],
        interpret=_INTERPRET,
    )(*args)
    return out, (ps2, pq2)


def _gcn2_body(pre_ref, ps_in, pq_in, g_ref, be_ref,
               p0_ref, ps0, pq0, g0_ref, be0_ref,
               a_ref, w_ref, out_ref, ps_ref, pq_ref):
    xn = _bn_relu(pre_ref[0], ps_in[...], pq_in[...], g_ref[...], be_ref[...])
    x0 = _bn_relu(p0_ref[0], ps0[...], pq0[...], g0_ref[...], be0_ref[...])
    h = jnp.dot(a_ref[0], xn, preferred_element_type=jnp.float32, precision=_HI)
    o = (1.0 - ALPHA) * h + ALPHA * x0
    o = (1.0 - BETA) * o + BETA * jnp.dot(o, w_ref[...], preferred_element_type=jnp.float32)
    out_ref[0] = o
    ps_ref[0], pq_ref[0] = _colsums(o)


def _gcn2_layer(pre, stats, gambet, pre0, stats0, gambet0, A, W):
    c = pre.shape[-1]
    in_specs = [pl.BlockSpec((1, NP, c), lambda g: (g, 0, 0)),
                _full((G, 1, c)), _full((G, 1, c)), _full((1, c)), _full((1, c)),
                pl.BlockSpec((1, NP, c), lambda g: (g, 0, 0)),
                _full((G, 1, c)), _full((G, 1, c)), _full((1, c)), _full((1, c)),
                pl.BlockSpec((1, NP, NP), lambda g: (g, 0, 0)),
                _full((c, c))]
    out, ps2, pq2 = pl.pallas_call(
        _gcn2_body,
        grid=(G,),
        in_specs=in_specs,
        out_specs=[pl.BlockSpec((1, NP, c), lambda g: (g, 0, 0)),
                   pl.BlockSpec((1, 1, c), lambda g: (g, 0, 0)),
                   pl.BlockSpec((1, 1, c), lambda g: (g, 0, 0))],
        out_shape=[_f32(G, NP, c), _f32(G, 1, c), _f32(G, 1, c)],
        interpret=_INTERPRET,
    )(pre, stats[0], stats[1], gambet[0], gambet[1],
      pre0, stats0[0], stats0[1], gambet0[0], gambet0[1], A, W)
    return out, (ps2, pq2)


def _prep_body(wacc_ref, cm_ref, aw_ref, a1_ref):
    rows = jax.lax.broadcasted_iota(jnp.int32, (NP, NP), 0)
    cols = jax.lax.broadcasted_iota(jnp.int32, (NP, NP), 1)
    eye = (rows == cols).astype(jnp.float32)
    for src, dst in ((wacc_ref, aw_ref), (cm_ref, a1_ref)):
        m = src[0] + eye
        deg = jnp.sum(m, axis=1)
        dis = 1.0 / jnp.sqrt(deg)
        dst[0] = m * dis[:, None] * dis[None, :]


def _prep_adj(wacc, cm):
    spec = pl.BlockSpec((1, NP, NP), lambda g: (g, 0, 0))
    return pl.pallas_call(
        _prep_body,
        grid=(G,),
        in_specs=[spec, spec],
        out_specs=[spec, spec],
        out_shape=[_f32(G, NP, NP), _f32(G, NP, NP)],
        interpret=_INTERPRET,
    )(wacc, cm)


def _head_body(pre_ref, ps_in, pq_in, g_ref, be_ref, alls_ref, adj_ref,
               p2w_ref, p2b_ref, ow_ref, ob_ref, out_ref):
    xn = _bn_relu(pre_ref[0], ps_in[...], pq_in[...], g_ref[...], be_ref[...])
    s = alls_ref[0]
    s = jnp.exp(s - jnp.max(s, axis=-1, keepdims=True))
    s = s / jnp.sum(s, axis=-1, keepdims=True)
    dn = (((0,), (0,)), ((), ()))
    nodes = jax.lax.dot_general(s, xn, dn, preferred_element_type=jnp.float32)
    adj = adj_ref[0]
    t1 = jnp.dot(adj, s, preferred_element_type=jnp.float32)
    oadj = jax.lax.dot_general(s, t1, dn, preferred_element_type=jnp.float32)
    eye5 = (jax.lax.broadcasted_iota(jnp.int32, (5, 5), 0)
            == jax.lax.broadcasted_iota(jnp.int32, (5, 5), 1)).astype(jnp.float32)
    a = oadj + eye5
    deg = jnp.clip(jnp.sum(a, axis=-1), 1.0, None)
    dis = 1.0 / jnp.sqrt(deg)
    an = a * dis[:, None] * dis[None, :]
    hw = jnp.dot(nodes, p2w_ref[...], preferred_element_type=jnp.float32)
    s2 = jnp.dot(an, hw, preferred_element_type=jnp.float32) + p2b_ref[0][None, :]
    s2 = jnp.exp(s2 - jnp.max(s2, axis=-1, keepdims=True))
    s2 = s2 / jnp.sum(s2, axis=-1, keepdims=True)
    xp = jax.lax.dot_general(s2, nodes, dn, preferred_element_type=jnp.float32)
    res = jnp.dot(xp, ow_ref[...], preferred_element_type=jnp.float32) + ob_ref[...]
    out_ref[0] = res


def _head(pre, stats, gambet, all_s, adj, p2w, p2b, ow, ob):
    in_specs = [pl.BlockSpec((1, NP, BSH), lambda g: (g, 0, 0)),
                _full((G, 1, BSH)), _full((G, 1, BSH)), _full((1, BSH)), _full((1, BSH)),
                pl.BlockSpec((1, NP, 5), lambda g: (g, 0, 0)),
                pl.BlockSpec((1, NP, NP), lambda g: (g, 0, 0)),
                _full((BSH, 1)), _full((1, 1)), _full((BSH, NOUT)), _full((1, NOUT))]
    return pl.pallas_call(
        _head_body,
        grid=(G,),
        in_specs=in_specs,
        out_specs=pl.BlockSpec((1, 1, NOUT), lambda g: (g, 0, 0)),
        out_shape=_f32(G, 1, NOUT),
        interpret=_INTERPRET,
    )(pre, stats[0], stats[1], gambet[0], gambet[1], all_s, adj,
      p2w, p2b.reshape(1, 1), ow, ob.reshape(1, NOUT))


def kernel(x, edge_index, edge_attr, params):
    P = params
    gb = lambda nm: (P[nm + '_bng'].reshape(1, -1), P[nm + '_bnb'].reshape(1, -1))

    off = (jnp.arange(G) * NP)[None, :, None]
    loc = edge_index.reshape(2, G, EP) - off
    eag = edge_attr.reshape(G, EP)
    gidx = jnp.arange(G)[:, None]
    wacc = jnp.zeros((G, NP, NP), jnp.float32).at[gidx, loc[1], loc[0]].add(eag)
    cm = jnp.zeros((G, NP, NP), jnp.float32).at[gidx, loc[1], loc[0]].add(1.0)
    adj = jnp.zeros((G, NP, NP), jnp.float32).at[gidx, loc[0], loc[1]].set(eag)
    aw, a1 = _prep_adj(wacc, cm)

    xg = x.reshape(G, NP, DIN)
    cur, st = _gcn_layer(xg, None, None, aw, P['enc0_W'], P['enc0_b'], first=True)
    saves = {}
    prev = 'enc0'
    for nm in ['enc1', 'enc2', 'enc3', 'enc4', 'enc5', 'enc6',
               'enc7', 'enc8', 'enc9', 'enc10', 'enc11']:
        cur, st = _gcn_layer(cur, st, gb(prev), aw, P[nm + '_W'], P[nm + '_b'])
        prev = nm
        if nm in ('enc2', 'enc5', 'enc8'):
            saves[nm] = (cur, st)

    def bn_relu_full(pre, st_, g_, b_):
        m = jnp.sum(st_[0][:, 0, :], 0) / N
        v = jnp.sum(st_[1][:, 0, :], 0) / N - m * m
        return jnp.maximum((pre.reshape(N, -1) - m) / jnp.sqrt(v + 1e-5) * g_[0] + b_[0], 0.0)

    def gat_jnp(xx, nm, h, o):
        loop = jnp.arange(N)
        s = jnp.concatenate([edge_index[0], loop])
        d = jnp.concatenate([edge_index[1], loop])
        xl = (xx @ P[nm + '_Wl']).reshape(N, h, o)
        xr = (xx @ P[nm + '_Wr']).reshape(N, h, o)
        e = jax.nn.leaky_relu(xl[s] + xr[d], 0.2)
        a = (e * P[nm + '_att'][None]).sum(-1)
        amax = jnp.full((N, h), -jnp.inf).at[d].max(a)
        ex = jnp.exp(a - amax[d])
        den = jnp.zeros((N, h)).at[d].add(ex)
        al = ex / (den[d] + 1e-16)
        out = jnp.zeros((N, h, o)).at[d].add(xl[s] * al[:, :, None])
        return (out.reshape(N, h * o) + P[nm + '_b']).reshape(G, NP, h * o)

    x4 = bn_relu_full(cur, st, *gb('enc11'))
    h4 = gat_jnp(x4, 'attg0', 2, 32)
    st4 = (jnp.sum(h4, axis=1)[:, None, :], jnp.sum(h4 * h4, axis=1)[:, None, :])

    cur, st = _gcn2_layer(h4, st4, gb('attg0'), *saves['enc8'], gb('enc8'), aw, P['attc2a_W'])
    cur, st = _gcn_layer(cur, st, gb('attc2a'), aw, P['attc0_W'], P['attc0_b'])

    x_in = bn_relu_full(cur, st, *gb('attc0'))
    h3 = gat_jnp(x_in, 'attg1', 2, 64)
    st3 = (jnp.sum(h3, axis=1)[:, None, :], jnp.sum(h3 * h3, axis=1)[:, None, :])

    cur, st = _gcn2_layer(h3, st3, gb('attg1'), *saves['enc5'], gb('enc5'), aw, P['decc2a_W'])
    cur, st = _gcn_layer(cur, st, gb('decc2a'), aw, P['dec0_W'], P['dec0_b'])
    cur, st = _gcn_layer(cur, st, gb('dec0'), a1, P['dec1_W'], P['dec1_b'])
    cur, st = _gcn2_layer(cur, st, gb('dec1'), *saves['enc2'], gb('enc2'), aw, P['decc2b_W'])
    cur, st = _gcn_layer(cur, st, gb('decc2b'), aw, P['dec2_W'], P['dec2_b'])
    cur, st = _gcn_layer(cur, st, gb('dec2'), aw, P['dec3_W'], P['dec3_b'])

    all_s, _ = _gcn_layer(cur, st, gb('dec3'), aw, P['pool1_W'], P['pool1_b'])
    return _head(cur, st, gb('dec3'), all_s, adj,
                 P['pool2_W'], P['pool2_b'], P['out_W'], P['out_b']).reshape(G, NOUT)


# speedup vs baseline: 24.7310x; 15.3910x over previous
import math

import jax
import jax.numpy as jnp
from jax.experimental import pallas as pl
from jax.experimental.pallas import tpu as pltpu

G = 32
NP = 512
EP = NP * 16
N = G * NP
E = G * EP
DIN = 128
BSH = 256
NOUT = 10
ALPHA = 0.5
BETA = math.log(0.1 / 2.0 + 1.0)

_INTERPRET = False

_HI = jax.lax.Precision.HIGHEST


def _f32(*shape):
    return jax.ShapeDtypeStruct(shape, jnp.float32)


def _bn_relu(pre, psum, psumsq, gam, bet):
    mean = jnp.sum(psum[:, 0, :], axis=0) / N
    msq = jnp.sum(psumsq[:, 0, :], axis=0) / N
    var = msq - mean * mean
    inv = gam[0] / jnp.sqrt(var + 1e-5)
    return jnp.maximum((pre - mean[None, :]) * inv[None, :] + bet[0][None, :], 0.0)


def _colsums(o):
    return jnp.sum(o, axis=0)[None, :], jnp.sum(o * o, axis=0)[None, :]


def _gcn_body(first):
    if first:
        def body(pre_ref, a_ref, w_ref, b_ref, out_ref, ps_ref, pq_ref):
            xn = pre_ref[0]
            h = jnp.dot(xn, w_ref[...], preferred_element_type=jnp.float32)
            o = jnp.dot(a_ref[0], h, preferred_element_type=jnp.float32,
                        precision=_HI) + b_ref[0][None, :]
            out_ref[0] = o
            ps_ref[0], pq_ref[0] = _colsums(o)
        return body

    def body(pre_ref, ps_in, pq_in, g_ref, be_ref, a_ref, w_ref, b_ref,
             out_ref, ps_ref, pq_ref):
        xn = _bn_relu(pre_ref[0], ps_in[...], pq_in[...], g_ref[...], be_ref[...])
        h = jnp.dot(xn, w_ref[...], preferred_element_type=jnp.float32)
        o = jnp.dot(a_ref[0], h, preferred_element_type=jnp.float32,
                    precision=_HI) + b_ref[0][None, :]
        out_ref[0] = o
        ps_ref[0], pq_ref[0] = _colsums(o)
    return body


def _full(shape):
    nd = len(shape)
    return pl.BlockSpec(shape, lambda g: (0,) * nd)


def _gcn_layer(pre, stats, gambet, A, W, b, first=False):
    cin = pre.shape[-1]
    cout = W.shape[-1]
    in_specs = [pl.BlockSpec((1, NP, cin), lambda g: (g, 0, 0))]
    args = [pre]
    if not first:
        ps, pq = stats
        gam, bet = gambet
        in_specs += [_full((G, 1, cin)), _full((G, 1, cin)),
                     _full((1, cin)), _full((1, cin))]
        args += [ps, pq, gam, bet]
    in_specs += [pl.BlockSpec((1, NP, NP), lambda g: (g, 0, 0)),
                 _full(W.shape), _full((1, cout))]
    args += [A, W, b.reshape(1, cout)]
    out, ps2, pq2 = pl.pallas_call(
        _gcn_body(first),
        grid=(G,),
        in_specs=in_specs,
        out_specs=[pl.BlockSpec((1, NP, cout), lambda g: (g, 0, 0)),
                   pl.BlockSpec((1, 1, cout), lambda g: (g, 0, 0)),
                   pl.BlockSpec((1, 1, cout), lambda g: (g, 0, 0))],
        out_shape=[_f32(G, NP, cout), _f32(G, 1, cout), _f32(G, 1, cout)],
        interpret=_INTERPRET,
    )(*args)
    return out, (ps2, pq2)


def _gcn2_body(pre_ref, ps_in, pq_in, g_ref, be_ref,
               p0_ref, ps0, pq0, g0_ref, be0_ref,
               a_ref, w_ref, out_ref, ps_ref, pq_ref):
    xn = _bn_relu(pre_ref[0], ps_in[...], pq_in[...], g_ref[...], be_ref[...])
    x0 = _bn_relu(p0_ref[0], ps0[...], pq0[...], g0_ref[...], be0_ref[...])
    h = jnp.dot(a_ref[0], xn, preferred_element_type=jnp.float32, precision=_HI)
    o = (1.0 - ALPHA) * h + ALPHA * x0
    o = (1.0 - BETA) * o + BETA * jnp.dot(o, w_ref[...], preferred_element_type=jnp.float32)
    out_ref[0] = o
    ps_ref[0], pq_ref[0] = _colsums(o)


def _gcn2_layer(pre, stats, gambet, pre0, stats0, gambet0, A, W):
    c = pre.shape[-1]
    in_specs = [pl.BlockSpec((1, NP, c), lambda g: (g, 0, 0)),
                _full((G, 1, c)), _full((G, 1, c)), _full((1, c)), _full((1, c)),
                pl.BlockSpec((1, NP, c), lambda g: (g, 0, 0)),
                _full((G, 1, c)), _full((G, 1, c)), _full((1, c)), _full((1, c)),
                pl.BlockSpec((1, NP, NP), lambda g: (g, 0, 0)),
                _full((c, c))]
    out, ps2, pq2 = pl.pallas_call(
        _gcn2_body,
        grid=(G,),
        in_specs=in_specs,
        out_specs=[pl.BlockSpec((1, NP, c), lambda g: (g, 0, 0)),
                   pl.BlockSpec((1, 1, c), lambda g: (g, 0, 0)),
                   pl.BlockSpec((1, 1, c), lambda g: (g, 0, 0))],
        out_shape=[_f32(G, NP, c), _f32(G, 1, c), _f32(G, 1, c)],
        interpret=_INTERPRET,
    )(pre, stats[0], stats[1], gambet[0], gambet[1],
      pre0, stats0[0], stats0[1], gambet0[0], gambet0[1], A, W)
    return out, (ps2, pq2)


def _gat_body(H, O, DT):
    def body(pre_ref, ps_in, pq_in, g_ref, be_ref, cm_ref,
             wl_ref, wr_ref, att_ref, b_ref, out_ref, ps_ref, pq_ref, xr_s):
        xn = _bn_relu(pre_ref[0], ps_in[...], pq_in[...], g_ref[...], be_ref[...])
        dnT = (((0,), (1,)), ((), ()))
        xl = jnp.dot(xn, wl_ref[...], preferred_element_type=jnp.float32)
        xlT = jax.lax.dot_general(wl_ref[...], xn, dnT,
                                  preferred_element_type=jnp.float32)
        xr_s[...] = jnp.dot(xn, wr_ref[...], preferred_element_type=jnp.float32)
        for h in range(H):
            xl_h = xl[:, h * O:(h + 1) * O]
            xlT_h = xlT[h * O:(h + 1) * O, :]
            att_h = att_ref[h, :].reshape(1, O, 1)

            def tile(i, _):
                d0 = i * DT
                xr_t = xr_s[pl.ds(d0, DT), h * O:(h + 1) * O].reshape(DT, O, 1)
                z = xlT_h[None, :, :] + xr_t
                z = jnp.where(z >= 0.0, z, 0.2 * z)
                s_t = jnp.sum(z * att_h, axis=1)
                m_t = cm_ref[0, pl.ds(d0, DT), :]
                rows = jax.lax.broadcasted_iota(jnp.int32, (DT, NP), 0) + d0
                cols = jax.lax.broadcasted_iota(jnp.int32, (DT, NP), 1)
                m_t = m_t + (rows == cols).astype(jnp.float32)
                live = m_t > 0.0
                amax = jnp.max(jnp.where(live, s_t, -jnp.inf), axis=1,
                               keepdims=True)
                ex = jnp.where(live, jnp.exp(s_t - amax), 0.0)
                den = jnp.sum(ex * m_t, axis=1, keepdims=True)
                alc = ex * m_t / (den + 1e-16)
                o_t = jnp.dot(alc, xl_h, preferred_element_type=jnp.float32,
                              precision=_HI)
                out_ref[0, pl.ds(d0, DT), h * O:(h + 1) * O] = (
                    o_t + b_ref[0][None, h * O:(h + 1) * O])
                return 0

            jax.lax.fori_loop(0, NP // DT, tile, 0)
        o = out_ref[0]
        ps_ref[0], pq_ref[0] = _colsums(o)
    return body


def _gat_layer(pre, stats, gambet, cm, Wl, Wr, att, b, H, O):
    cin = pre.shape[-1]
    cout = H * O
    in_specs = [pl.BlockSpec((1, NP, cin), lambda g: (g, 0, 0)),
                _full((G, 1, cin)), _full((G, 1, cin)),
                _full((1, cin)), _full((1, cin)),
                pl.BlockSpec((1, NP, NP), lambda g: (g, 0, 0)),
                _full((cin, cout)), _full((cin, cout)),
                _full((H, O)), _full((1, cout))]
    out, ps2, pq2 = pl.pallas_call(
        _gat_body(H, O, 32),
        grid=(G,),
        in_specs=in_specs,
        out_specs=[pl.BlockSpec((1, NP, cout), lambda g: (g, 0, 0)),
                   pl.BlockSpec((1, 1, cout), lambda g: (g, 0, 0)),
                   pl.BlockSpec((1, 1, cout), lambda g: (g, 0, 0))],
        out_shape=[_f32(G, NP, cout), _f32(G, 1, cout), _f32(G, 1, cout)],
        scratch_shapes=[pltpu.VMEM((NP, cout), jnp.float32)],
        interpret=_INTERPRET,
    )(pre, stats[0], stats[1], gambet[0], gambet[1], cm, Wl, Wr, att,
      b.reshape(1, cout))
    return out, (ps2, pq2)


def _prep_body(wacc_ref, cm_ref, aw_ref, a1_ref):
    rows = jax.lax.broadcasted_iota(jnp.int32, (NP, NP), 0)
    cols = jax.lax.broadcasted_iota(jnp.int32, (NP, NP), 1)
    eye = (rows == cols).astype(jnp.float32)
    for src, dst in ((wacc_ref, aw_ref), (cm_ref, a1_ref)):
        m = src[0] + eye
        deg = jnp.sum(m, axis=1)
        dis = 1.0 / jnp.sqrt(deg)
        dst[0] = m * dis[:, None] * dis[None, :]


def _prep_adj(wacc, cm):
    spec = pl.BlockSpec((1, NP, NP), lambda g: (g, 0, 0))
    return pl.pallas_call(
        _prep_body,
        grid=(G,),
        in_specs=[spec, spec],
        out_specs=[spec, spec],
        out_shape=[_f32(G, NP, NP), _f32(G, NP, NP)],
        interpret=_INTERPRET,
    )(wacc, cm)


def _head_body(pre_ref, ps_in, pq_in, g_ref, be_ref, alls_ref, adj_ref,
               p2w_ref, p2b_ref, ow_ref, ob_ref, out_ref):
    xn = _bn_relu(pre_ref[0], ps_in[...], pq_in[...], g_ref[...], be_ref[...])
    s = alls_ref[0]
    s = jnp.exp(s - jnp.max(s, axis=-1, keepdims=True))
    s = s / jnp.sum(s, axis=-1, keepdims=True)
    dn = (((0,), (0,)), ((), ()))
    nodes = jax.lax.dot_general(s, xn, dn, preferred_element_type=jnp.float32)
    adj = adj_ref[0]
    t1 = jnp.dot(adj, s, preferred_element_type=jnp.float32)
    oadj = jax.lax.dot_general(s, t1, dn, preferred_element_type=jnp.float32)
    eye5 = (jax.lax.broadcasted_iota(jnp.int32, (5, 5), 0)
            == jax.lax.broadcasted_iota(jnp.int32, (5, 5), 1)).astype(jnp.float32)
    a = oadj + eye5
    deg = jnp.clip(jnp.sum(a, axis=-1), 1.0, None)
    dis = 1.0 / jnp.sqrt(deg)
    an = a * dis[:, None] * dis[None, :]
    hw = jnp.dot(nodes, p2w_ref[...], preferred_element_type=jnp.float32)
    s2 = jnp.dot(an, hw, preferred_element_type=jnp.float32) + p2b_ref[0][None, :]
    s2 = jnp.exp(s2 - jnp.max(s2, axis=-1, keepdims=True))
    s2 = s2 / jnp.sum(s2, axis=-1, keepdims=True)
    xp = jax.lax.dot_general(s2, nodes, dn, preferred_element_type=jnp.float32)
    res = jnp.dot(xp, ow_ref[...], preferred_element_type=jnp.float32) + ob_ref[...]
    out_ref[0] = res


def _head(pre, stats, gambet, all_s, adj, p2w, p2b, ow, ob):
    in_specs = [pl.BlockSpec((1, NP, BSH), lambda g: (g, 0, 0)),
                _full((G, 1, BSH)), _full((G, 1, BSH)), _full((1, BSH)), _full((1, BSH)),
                pl.BlockSpec((1, NP, 5), lambda g: (g, 0, 0)),
                pl.BlockSpec((1, NP, NP), lambda g: (g, 0, 0)),
                _full((BSH, 1)), _full((1, 1)), _full((BSH, NOUT)), _full((1, NOUT))]
    return pl.pallas_call(
        _head_body,
        grid=(G,),
        in_specs=in_specs,
        out_specs=pl.BlockSpec((1, 1, NOUT), lambda g: (g, 0, 0)),
        out_shape=_f32(G, 1, NOUT),
        interpret=_INTERPRET,
    )(pre, stats[0], stats[1], gambet[0], gambet[1], all_s, adj,
      p2w, p2b.reshape(1, 1), ow, ob.reshape(1, NOUT))


def kernel(x, edge_index, edge_attr, params):
    P = params
    gb = lambda nm: (P[nm + '_bng'].reshape(1, -1), P[nm + '_bnb'].reshape(1, -1))

    off = (jnp.arange(G) * NP)[None, :, None]
    loc = edge_index.reshape(2, G, EP) - off
    eag = edge_attr.reshape(G, EP)
    gidx = jnp.arange(G)[:, None]
    wacc = jnp.zeros((G, NP, NP), jnp.float32).at[gidx, loc[1], loc[0]].add(eag)
    cm = jnp.zeros((G, NP, NP), jnp.float32).at[gidx, loc[1], loc[0]].add(1.0)
    adj = jnp.zeros((G, NP, NP), jnp.float32).at[gidx, loc[0], loc[1]].set(eag)
    aw, a1 = _prep_adj(wacc, cm)

    xg = x.reshape(G, NP, DIN)
    cur, st = _gcn_layer(xg, None, None, aw, P['enc0_W'], P['enc0_b'], first=True)
    saves = {}
    prev = 'enc0'
    for nm in ['enc1', 'enc2', 'enc3', 'enc4', 'enc5', 'enc6',
               'enc7', 'enc8', 'enc9', 'enc10', 'enc11']:
        cur, st = _gcn_layer(cur, st, gb(prev), aw, P[nm + '_W'], P[nm + '_b'])
        prev = nm
        if nm in ('enc2', 'enc5', 'enc8'):
            saves[nm] = (cur, st)
    cur, st = _gat_layer(cur, st, gb('enc11'), cm, P['attg0_Wl'], P['attg0_Wr'],
                         P['attg0_att'], P['attg0_b'], 2, 32)
    cur, st = _gcn2_layer(cur, st, gb('attg0'), *saves['enc8'], gb('enc8'), aw, P['attc2a_W'])
    cur, st = _gcn_layer(cur, st, gb('attc2a'), aw, P['attc0_W'], P['attc0_b'])
    cur, st = _gat_layer(cur, st, gb('attc0'), cm, P['attg1_Wl'], P['attg1_Wr'],
                         P['attg1_att'], P['attg1_b'], 2, 64)
    cur, st = _gcn2_layer(cur, st, gb('attg1'), *saves['enc5'], gb('enc5'), aw, P['decc2a_W'])
    cur, st = _gcn_layer(cur, st, gb('decc2a'), aw, P['dec0_W'], P['dec0_b'])
    cur, st = _gcn_layer(cur, st, gb('dec0'), a1, P['dec1_W'], P['dec1_b'])
    cur, st = _gcn2_layer(cur, st, gb('dec1'), *saves['enc2'], gb('enc2'), aw, P['decc2b_W'])
    cur, st = _gcn_layer(cur, st, gb('decc2b'), aw, P['dec2_W'], P['dec2_b'])
    cur, st = _gcn_layer(cur, st, gb('dec2'), aw, P['dec3_W'], P['dec3_b'])

    all_s, _ = _gcn_layer(cur, st, gb('dec3'), aw, P['pool1_W'], P['pool1_b'])
    return _head(cur, st, gb('dec3'), all_s, adj,
                 P['pool2_W'], P['pool2_b'], P['out_W'], P['out_b']).reshape(G, NOUT)


# speedup vs baseline: 30.7710x; 1.2442x over previous
import functools
import math

import jax
import jax.numpy as jnp
from jax import lax
from jax.experimental import pallas as pl
from jax.experimental.pallas import tpu as pltpu
from jax.experimental.pallas import tpu_sc as plsc

G = 32
NP = 512
EP = NP * 16
N = G * NP
E = G * EP
DIN = 128
BSH = 256
NOUT = 10
ALPHA = 0.5
BETA = math.log(0.1 / 2.0 + 1.0)

_INTERPRET = False

_HI = jax.lax.Precision.HIGHEST


def _f32(*shape):
    return jax.ShapeDtypeStruct(shape, jnp.float32)


def _bn_relu(pre, psum, psumsq, gam, bet):
    mean = jnp.sum(psum[:, 0, :], axis=0) / N
    msq = jnp.sum(psumsq[:, 0, :], axis=0) / N
    var = msq - mean * mean
    inv = gam[0] / jnp.sqrt(var + 1e-5)
    return jnp.maximum((pre - mean[None, :]) * inv[None, :] + bet[0][None, :], 0.0)


def _colsums(o):
    return jnp.sum(o, axis=0)[None, :], jnp.sum(o * o, axis=0)[None, :]


def _gcn_body(first):
    if first:
        def body(pre_ref, a_ref, w_ref, b_ref, out_ref, ps_ref, pq_ref):
            xn = pre_ref[0]
            h = jnp.dot(xn, w_ref[...], preferred_element_type=jnp.float32)
            o = jnp.dot(a_ref[0], h, preferred_element_type=jnp.float32,
                        precision=_HI) + b_ref[0][None, :]
            out_ref[0] = o
            ps_ref[0], pq_ref[0] = _colsums(o)
        return body

    def body(pre_ref, ps_in, pq_in, g_ref, be_ref, a_ref, w_ref, b_ref,
             out_ref, ps_ref, pq_ref):
        xn = _bn_relu(pre_ref[0], ps_in[...], pq_in[...], g_ref[...], be_ref[...])
        h = jnp.dot(xn, w_ref[...], preferred_element_type=jnp.float32)
        o = jnp.dot(a_ref[0], h, preferred_element_type=jnp.float32,
                    precision=_HI) + b_ref[0][None, :]
        out_ref[0] = o
        ps_ref[0], pq_ref[0] = _colsums(o)
    return body


def _full(shape):
    nd = len(shape)
    return pl.BlockSpec(shape, lambda g: (0,) * nd)


def _gcn_layer(pre, stats, gambet, A, W, b, first=False):
    cin = pre.shape[-1]
    cout = W.shape[-1]
    in_specs = [pl.BlockSpec((1, NP, cin), lambda g: (g, 0, 0))]
    args = [pre]
    if not first:
        ps, pq = stats
        gam, bet = gambet
        in_specs += [_full((G, 1, cin)), _full((G, 1, cin)),
                     _full((1, cin)), _full((1, cin))]
        args += [ps, pq, gam, bet]
    in_specs += [pl.BlockSpec((1, NP, NP), lambda g: (g, 0, 0)),
                 _full(W.shape), _full((1, cout))]
    args += [A, W, b.reshape(1, cout)]
    out, ps2, pq2 = pl.pallas_call(
        _gcn_body(first),
        grid=(G,),
        in_specs=in_specs,
        out_specs=[pl.BlockSpec((1, NP, cout), lambda g: (g, 0, 0)),
                   pl.BlockSpec((1, 1, cout), lambda g: (g, 0, 0)),
                   pl.BlockSpec((1, 1, cout), lambda g: (g, 0, 0))],
        out_shape=[_f32(G, NP, cout), _f32(G, 1, cout), _f32(G, 1, cout)],
        interpret=_INTERPRET,
    )(*args)
    return out, (ps2, pq2)


def _gcn2_body(pre_ref, ps_in, pq_in, g_ref, be_ref,
               p0_ref, ps0, pq0, g0_ref, be0_ref,
               a_ref, w_ref, out_ref, ps_ref, pq_ref):
    xn = _bn_relu(pre_ref[0], ps_in[...], pq_in[...], g_ref[...], be_ref[...])
    x0 = _bn_relu(p0_ref[0], ps0[...], pq0[...], g0_ref[...], be0_ref[...])
    h = jnp.dot(a_ref[0], xn, preferred_element_type=jnp.float32, precision=_HI)
    o = (1.0 - ALPHA) * h + ALPHA * x0
    o = (1.0 - BETA) * o + BETA * jnp.dot(o, w_ref[...], preferred_element_type=jnp.float32)
    out_ref[0] = o
    ps_ref[0], pq_ref[0] = _colsums(o)


def _gcn2_layer(pre, stats, gambet, pre0, stats0, gambet0, A, W):
    c = pre.shape[-1]
    in_specs = [pl.BlockSpec((1, NP, c), lambda g: (g, 0, 0)),
                _full((G, 1, c)), _full((G, 1, c)), _full((1, c)), _full((1, c)),
                pl.BlockSpec((1, NP, c), lambda g: (g, 0, 0)),
                _full((G, 1, c)), _full((G, 1, c)), _full((1, c)), _full((1, c)),
                pl.BlockSpec((1, NP, NP), lambda g: (g, 0, 0)),
                _full((c, c))]
    out, ps2, pq2 = pl.pallas_call(
        _gcn2_body,
        grid=(G,),
        in_specs=in_specs,
        out_specs=[pl.BlockSpec((1, NP, c), lambda g: (g, 0, 0)),
                   pl.BlockSpec((1, 1, c), lambda g: (g, 0, 0)),
                   pl.BlockSpec((1, 1, c), lambda g: (g, 0, 0))],
        out_shape=[_f32(G, NP, c), _f32(G, 1, c), _f32(G, 1, c)],
        interpret=_INTERPRET,
    )(pre, stats[0], stats[1], gambet[0], gambet[1],
      pre0, stats0[0], stats0[1], gambet0[0], gambet0[1], A, W)
    return out, (ps2, pq2)


def _gat_body(H, O, DT):
    def body(pre_ref, ps_in, pq_in, g_ref, be_ref, cm_ref,
             wl_ref, wr_ref, att_ref, b_ref, out_ref, ps_ref, pq_ref, xr_s):
        xn = _bn_relu(pre_ref[0], ps_in[...], pq_in[...], g_ref[...], be_ref[...])
        dnT = (((0,), (1,)), ((), ()))
        xl = jnp.dot(xn, wl_ref[...], preferred_element_type=jnp.float32)
        xlT = jax.lax.dot_general(wl_ref[...], xn, dnT,
                                  preferred_element_type=jnp.float32)
        xr_s[...] = jnp.dot(xn, wr_ref[...], preferred_element_type=jnp.float32)
        for h in range(H):
            xl_h = xl[:, h * O:(h + 1) * O]
            xlT_h = xlT[h * O:(h + 1) * O, :]
            att_h = att_ref[h, :].reshape(1, O, 1)

            def tile(i, _):
                d0 = i * DT
                xr_t = xr_s[pl.ds(d0, DT), h * O:(h + 1) * O].reshape(DT, O, 1)
                z = xlT_h[None, :, :] + xr_t
                z = jnp.where(z >= 0.0, z, 0.2 * z)
                s_t = jnp.sum(z * att_h, axis=1)
                m_t = cm_ref[0, pl.ds(d0, DT), :]
                rows = jax.lax.broadcasted_iota(jnp.int32, (DT, NP), 0) + d0
                cols = jax.lax.broadcasted_iota(jnp.int32, (DT, NP), 1)
                m_t = m_t + (rows == cols).astype(jnp.float32)
                live = m_t > 0.0
                amax = jnp.max(jnp.where(live, s_t, -jnp.inf), axis=1,
                               keepdims=True)
                ex = jnp.where(live, jnp.exp(s_t - amax), 0.0)
                den = jnp.sum(ex * m_t, axis=1, keepdims=True)
                alc = ex * m_t / (den + 1e-16)
                o_t = jnp.dot(alc, xl_h, preferred_element_type=jnp.float32,
                              precision=_HI)
                out_ref[0, pl.ds(d0, DT), h * O:(h + 1) * O] = (
                    o_t + b_ref[0][None, h * O:(h + 1) * O])
                return 0

            jax.lax.fori_loop(0, NP // DT, tile, 0)
        o = out_ref[0]
        ps_ref[0], pq_ref[0] = _colsums(o)
    return body


def _gat_layer(pre, stats, gambet, cm, Wl, Wr, att, b, H, O):
    cin = pre.shape[-1]
    cout = H * O
    in_specs = [pl.BlockSpec((1, NP, cin), lambda g: (g, 0, 0)),
                _full((G, 1, cin)), _full((G, 1, cin)),
                _full((1, cin)), _full((1, cin)),
                pl.BlockSpec((1, NP, NP), lambda g: (g, 0, 0)),
                _full((cin, cout)), _full((cin, cout)),
                _full((H, O)), _full((1, cout))]
    out, ps2, pq2 = pl.pallas_call(
        _gat_body(H, O, 32),
        grid=(G,),
        in_specs=in_specs,
        out_specs=[pl.BlockSpec((1, NP, cout), lambda g: (g, 0, 0)),
                   pl.BlockSpec((1, 1, cout), lambda g: (g, 0, 0)),
                   pl.BlockSpec((1, 1, cout), lambda g: (g, 0, 0))],
        out_shape=[_f32(G, NP, cout), _f32(G, 1, cout), _f32(G, 1, cout)],
        scratch_shapes=[pltpu.VMEM((NP, cout), jnp.float32)],
        interpret=_INTERPRET,
    )(pre, stats[0], stats[1], gambet[0], gambet[1], cm, Wl, Wr, att,
      b.reshape(1, cout))
    return out, (ps2, pq2)


_RB = 64
_BIG = 1 << 24


def _adj_build(ei, ea):
    mesh = plsc.VectorSubcoreMesh(core_axis_name="c", subcore_axis_name="s")

    @functools.partial(
        pl.kernel, mesh=mesh,
        compiler_params=pltpu.CompilerParams(needs_layout_passes=False),
        out_type=[jax.ShapeDtypeStruct((G, NP * NP), jnp.float32)] * 3,
        scratch_types=[
            pltpu.VMEM((EP,), jnp.int32),
            pltpu.VMEM((EP,), jnp.int32),
            pltpu.VMEM((EP,), jnp.float32),
            pltpu.VMEM((_RB * NP,), jnp.float32),
            pltpu.VMEM((_RB * NP,), jnp.float32),
            pltpu.VMEM((_RB * NP,), jnp.float32),
        ])
    def k(ei_hbm, ea_hbm, wacc_hbm, cnt_hbm, adj_hbm, s_v, d_v, w_v, wb, cb, ab):
        t = lax.axis_index("s") * 2 + lax.axis_index("c")
        pltpu.sync_copy(ei_hbm.at[0, pl.ds(t * EP, EP)], s_v)
        pltpu.sync_copy(ei_hbm.at[1, pl.ds(t * EP, EP)], d_v)
        pltpu.sync_copy(ea_hbm.at[pl.ds(t * EP, EP)], w_v)
        base = t * NP
        lane = lax.iota(jnp.int32, 16)
        z16 = jnp.zeros((16,), jnp.float32)
        one16 = jnp.ones((16,), jnp.float32)
        for r in range(NP // _RB):
            def zero(i, c):
                wb[pl.ds(i * 16, 16)] = z16
                cb[pl.ds(i * 16, 16)] = z16
                ab[pl.ds(i * 16, 16)] = z16
                return c
            lax.fori_loop(0, _RB * NP // 16, zero, 0)
            r0 = r * _RB

            def vec(v, c):
                sg = s_v[pl.ds(v * 16, 16)]
                dg = d_v[pl.ds(v * 16, 16)]
                w = w_v[pl.ds(v * 16, 16)]
                dl = dg - (base + r0)
                sl = sg - base
                valid = (dl >= 0) & (dl < _RB)
                idx = jnp.where(valid, dl * NP + sl, 0)
                plsc.addupdate_scatter(wb, [idx], w, mask=valid)
                plsc.addupdate_scatter(cb, [idx], one16, mask=valid)
                plsc.store_scatter(ab, [idx], w, mask=valid)
                return c
            lax.fori_loop(0, EP // 16, vec, 0)

            def fix(v, c):
                sg = s_v[pl.ds(v * 16, 16)]
                dg = d_v[pl.ds(v * 16, 16)]
                w = w_v[pl.ds(v * 16, 16)]
                dl = dg - (base + r0)
                sl = sg - base
                valid = (dl >= 0) & (dl < _RB)
                idx = jnp.where(valid, dl * NP + sl, 0)
                cnt = plsc.load_gather(cb, [idx], mask=valid)
                flg = valid & (cnt >= 2.0)
                nfl = jnp.max(plsc.all_reduce_population_count(flg))

                @pl.when(nfl > 0)
                def _():
                    for l in range(16):
                        plsc.store_scatter(ab, [idx], w, mask=flg & (lane == l))
                return c
            lax.fori_loop(0, EP // 16, fix, 0)
            pltpu.sync_copy(wb, wacc_hbm.at[t, pl.ds(r0 * NP, _RB * NP)])
            pltpu.sync_copy(cb, cnt_hbm.at[t, pl.ds(r0 * NP, _RB * NP)])
            pltpu.sync_copy(ab, adj_hbm.at[t, pl.ds(r0 * NP, _RB * NP)])

    wacc, cnt, adjt = k(ei, ea)
    return (wacc.reshape(G, NP, NP), cnt.reshape(G, NP, NP),
            adjt.reshape(G, NP, NP))


def _prep_body(wacc_ref, cm_ref, aw_ref, a1_ref):
    rows = jax.lax.broadcasted_iota(jnp.int32, (NP, NP), 0)
    cols = jax.lax.broadcasted_iota(jnp.int32, (NP, NP), 1)
    eye = (rows == cols).astype(jnp.float32)
    for src, dst in ((wacc_ref, aw_ref), (cm_ref, a1_ref)):
        m = src[0] + eye
        deg = jnp.sum(m, axis=1)
        dis = 1.0 / jnp.sqrt(deg)
        dst[0] = m * dis[:, None] * dis[None, :]


def _prep_adj(wacc, cm):
    spec = pl.BlockSpec((1, NP, NP), lambda g: (g, 0, 0))
    return pl.pallas_call(
        _prep_body,
        grid=(G,),
        in_specs=[spec, spec],
        out_specs=[spec, spec],
        out_shape=[_f32(G, NP, NP), _f32(G, NP, NP)],
        interpret=_INTERPRET,
    )(wacc, cm)


def _head_body(pre_ref, ps_in, pq_in, g_ref, be_ref, alls_ref, adj_ref,
               p2w_ref, p2b_ref, ow_ref, ob_ref, out_ref):
    xn = _bn_relu(pre_ref[0], ps_in[...], pq_in[...], g_ref[...], be_ref[...])
    s = alls_ref[0]
    s = jnp.exp(s - jnp.max(s, axis=-1, keepdims=True))
    s = s / jnp.sum(s, axis=-1, keepdims=True)
    dn = (((0,), (0,)), ((), ()))
    nodes = jax.lax.dot_general(s, xn, dn, preferred_element_type=jnp.float32)
    adjt = adj_ref[0]
    t1 = jax.lax.dot_general(adjt, s, dn, preferred_element_type=jnp.float32)
    oadj = jax.lax.dot_general(s, t1, dn, preferred_element_type=jnp.float32)
    eye5 = (jax.lax.broadcasted_iota(jnp.int32, (5, 5), 0)
            == jax.lax.broadcasted_iota(jnp.int32, (5, 5), 1)).astype(jnp.float32)
    a = oadj + eye5
    deg = jnp.clip(jnp.sum(a, axis=-1), 1.0, None)
    dis = 1.0 / jnp.sqrt(deg)
    an = a * dis[:, None] * dis[None, :]
    hw = jnp.dot(nodes, p2w_ref[...], preferred_element_type=jnp.float32)
    s2 = jnp.dot(an, hw, preferred_element_type=jnp.float32) + p2b_ref[0][None, :]
    s2 = jnp.exp(s2 - jnp.max(s2, axis=-1, keepdims=True))
    s2 = s2 / jnp.sum(s2, axis=-1, keepdims=True)
    xp = jax.lax.dot_general(s2, nodes, dn, preferred_element_type=jnp.float32)
    res = jnp.dot(xp, ow_ref[...], preferred_element_type=jnp.float32) + ob_ref[...]
    out_ref[0] = res


def _head(pre, stats, gambet, all_s, adj, p2w, p2b, ow, ob):
    in_specs = [pl.BlockSpec((1, NP, BSH), lambda g: (g, 0, 0)),
                _full((G, 1, BSH)), _full((G, 1, BSH)), _full((1, BSH)), _full((1, BSH)),
                pl.BlockSpec((1, NP, 5), lambda g: (g, 0, 0)),
                pl.BlockSpec((1, NP, NP), lambda g: (g, 0, 0)),
                _full((BSH, 1)), _full((1, 1)), _full((BSH, NOUT)), _full((1, NOUT))]
    return pl.pallas_call(
        _head_body,
        grid=(G,),
        in_specs=in_specs,
        out_specs=pl.BlockSpec((1, 1, NOUT), lambda g: (g, 0, 0)),
        out_shape=_f32(G, 1, NOUT),
        interpret=_INTERPRET,
    )(pre, stats[0], stats[1], gambet[0], gambet[1], all_s, adj,
      p2w, p2b.reshape(1, 1), ow, ob.reshape(1, NOUT))


def kernel(x, edge_index, edge_attr, params):
    P = params
    gb = lambda nm: (P[nm + '_bng'].reshape(1, -1), P[nm + '_bnb'].reshape(1, -1))

    wacc, cm, adjt = _adj_build(edge_index, edge_attr)
    aw, a1 = _prep_adj(wacc, cm)

    xg = x.reshape(G, NP, DIN)
    cur, st = _gcn_layer(xg, None, None, aw, P['enc0_W'], P['enc0_b'], first=True)
    saves = {}
    prev = 'enc0'
    for nm in ['enc1', 'enc2', 'enc3', 'enc4', 'enc5', 'enc6',
               'enc7', 'enc8', 'enc9', 'enc10', 'enc11']:
        cur, st = _gcn_layer(cur, st, gb(prev), aw, P[nm + '_W'], P[nm + '_b'])
        prev = nm
        if nm in ('enc2', 'enc5', 'enc8'):
            saves[nm] = (cur, st)
    cur, st = _gat_layer(cur, st, gb('enc11'), cm, P['attg0_Wl'], P['attg0_Wr'],
                         P['attg0_att'], P['attg0_b'], 2, 32)
    cur, st = _gcn2_layer(cur, st, gb('attg0'), *saves['enc8'], gb('enc8'), aw, P['attc2a_W'])
    cur, st = _gcn_layer(cur, st, gb('attc2a'), aw, P['attc0_W'], P['attc0_b'])
    cur, st = _gat_layer(cur, st, gb('attc0'), cm, P['attg1_Wl'], P['attg1_Wr'],
                         P['attg1_att'], P['attg1_b'], 2, 64)
    cur, st = _gcn2_layer(cur, st, gb('attg1'), *saves['enc5'], gb('enc5'), aw, P['decc2a_W'])
    cur, st = _gcn_layer(cur, st, gb('decc2a'), aw, P['dec0_W'], P['dec0_b'])
    cur, st = _gcn_layer(cur, st, gb('dec0'), a1, P['dec1_W'], P['dec1_b'])
    cur, st = _gcn2_layer(cur, st, gb('dec1'), *saves['enc2'], gb('enc2'), aw, P['decc2b_W'])
    cur, st = _gcn_layer(cur, st, gb('decc2b'), aw, P['dec2_W'], P['dec2_b'])
    cur, st = _gcn_layer(cur, st, gb('dec2'), aw, P['dec3_W'], P['dec3_b'])

    all_s, _ = _gcn_layer(cur, st, gb('dec3'), aw, P['pool1_W'], P['pool1_b'])
    return _head(cur, st, gb('dec3'), all_s, adjt,
                 P['pool2_W'], P['pool2_b'], P['out_W'], P['out_b']).reshape(G, NOUT)


# speedup vs baseline: 31.1310x; 1.0117x over previous
import functools
import math

import jax
import jax.numpy as jnp
from jax import lax
from jax.experimental import pallas as pl
from jax.experimental.pallas import tpu as pltpu
from jax.experimental.pallas import tpu_sc as plsc

G = 32
NP = 512
EP = NP * 16
N = G * NP
E = G * EP
DIN = 128
BSH = 256
NOUT = 10
ALPHA = 0.5
BETA = math.log(0.1 / 2.0 + 1.0)

_INTERPRET = False

_HI = jax.lax.Precision.HIGHEST


def _f32(*shape):
    return jax.ShapeDtypeStruct(shape, jnp.float32)


def _bn_relu(pre, psum, psumsq, gam, bet):
    mean = jnp.sum(psum[:, 0, :], axis=0) / N
    msq = jnp.sum(psumsq[:, 0, :], axis=0) / N
    var = msq - mean * mean
    inv = gam[0] / jnp.sqrt(var + 1e-5)
    return jnp.maximum((pre - mean[None, :]) * inv[None, :] + bet[0][None, :], 0.0)


def _colsums(o):
    return jnp.sum(o, axis=0)[None, :], jnp.sum(o * o, axis=0)[None, :]


def _gcn_body(first):
    if first:
        def body(pre_ref, a_ref, w_ref, b_ref, out_ref, ps_ref, pq_ref):
            xn = pre_ref[0]
            h = jnp.dot(xn, w_ref[...], preferred_element_type=jnp.float32)
            o = jnp.dot(a_ref[0], h, preferred_element_type=jnp.float32,
                        precision=_HI) + b_ref[0][None, :]
            out_ref[0] = o
            ps_ref[0], pq_ref[0] = _colsums(o)
        return body

    def body(pre_ref, ps_in, pq_in, g_ref, be_ref, a_ref, w_ref, b_ref,
             out_ref, ps_ref, pq_ref):
        xn = _bn_relu(pre_ref[0], ps_in[...], pq_in[...], g_ref[...], be_ref[...])
        h = jnp.dot(xn, w_ref[...], preferred_element_type=jnp.float32)
        o = jnp.dot(a_ref[0], h, preferred_element_type=jnp.float32,
                    precision=_HI) + b_ref[0][None, :]
        out_ref[0] = o
        ps_ref[0], pq_ref[0] = _colsums(o)
    return body


def _full(shape):
    nd = len(shape)
    return pl.BlockSpec(shape, lambda g: (0,) * nd)


def _gcn_layer(pre, stats, gambet, A, W, b, first=False):
    cin = pre.shape[-1]
    cout = W.shape[-1]
    in_specs = [pl.BlockSpec((1, NP, cin), lambda g: (g, 0, 0))]
    args = [pre]
    if not first:
        ps, pq = stats
        gam, bet = gambet
        in_specs += [_full((G, 1, cin)), _full((G, 1, cin)),
                     _full((1, cin)), _full((1, cin))]
        args += [ps, pq, gam, bet]
    in_specs += [pl.BlockSpec((1, NP, NP), lambda g: (g, 0, 0)),
                 _full(W.shape), _full((1, cout))]
    args += [A, W, b.reshape(1, cout)]
    out, ps2, pq2 = pl.pallas_call(
        _gcn_body(first),
        grid=(G,),
        in_specs=in_specs,
        out_specs=[pl.BlockSpec((1, NP, cout), lambda g: (g, 0, 0)),
                   pl.BlockSpec((1, 1, cout), lambda g: (g, 0, 0)),
                   pl.BlockSpec((1, 1, cout), lambda g: (g, 0, 0))],
        out_shape=[_f32(G, NP, cout), _f32(G, 1, cout), _f32(G, 1, cout)],
        interpret=_INTERPRET,
    )(*args)
    return out, (ps2, pq2)


def _gcn2_body(pre_ref, ps_in, pq_in, g_ref, be_ref,
               p0_ref, ps0, pq0, g0_ref, be0_ref,
               a_ref, w_ref, out_ref, ps_ref, pq_ref):
    xn = _bn_relu(pre_ref[0], ps_in[...], pq_in[...], g_ref[...], be_ref[...])
    x0 = _bn_relu(p0_ref[0], ps0[...], pq0[...], g0_ref[...], be0_ref[...])
    h = jnp.dot(a_ref[0], xn, preferred_element_type=jnp.float32, precision=_HI)
    o = (1.0 - ALPHA) * h + ALPHA * x0
    o = (1.0 - BETA) * o + BETA * jnp.dot(o, w_ref[...], preferred_element_type=jnp.float32)
    out_ref[0] = o
    ps_ref[0], pq_ref[0] = _colsums(o)


def _gcn2_layer(pre, stats, gambet, pre0, stats0, gambet0, A, W):
    c = pre.shape[-1]
    in_specs = [pl.BlockSpec((1, NP, c), lambda g: (g, 0, 0)),
                _full((G, 1, c)), _full((G, 1, c)), _full((1, c)), _full((1, c)),
                pl.BlockSpec((1, NP, c), lambda g: (g, 0, 0)),
                _full((G, 1, c)), _full((G, 1, c)), _full((1, c)), _full((1, c)),
                pl.BlockSpec((1, NP, NP), lambda g: (g, 0, 0)),
                _full((c, c))]
    out, ps2, pq2 = pl.pallas_call(
        _gcn2_body,
        grid=(G,),
        in_specs=in_specs,
        out_specs=[pl.BlockSpec((1, NP, c), lambda g: (g, 0, 0)),
                   pl.BlockSpec((1, 1, c), lambda g: (g, 0, 0)),
                   pl.BlockSpec((1, 1, c), lambda g: (g, 0, 0))],
        out_shape=[_f32(G, NP, c), _f32(G, 1, c), _f32(G, 1, c)],
        interpret=_INTERPRET,
    )(pre, stats[0], stats[1], gambet[0], gambet[1],
      pre0, stats0[0], stats0[1], gambet0[0], gambet0[1], A, W)
    return out, (ps2, pq2)


def _gat_body(H, O, DT):
    def body(pre_ref, ps_in, pq_in, g_ref, be_ref, cm_ref,
             wl_ref, wr_ref, att_ref, b_ref, out_ref, ps_ref, pq_ref, xr_s):
        xn = _bn_relu(pre_ref[0], ps_in[...], pq_in[...], g_ref[...], be_ref[...])
        dnT = (((0,), (1,)), ((), ()))
        xl = jnp.dot(xn, wl_ref[...], preferred_element_type=jnp.float32)
        xlT = jax.lax.dot_general(wl_ref[...], xn, dnT,
                                  preferred_element_type=jnp.float32)
        xr_s[...] = jnp.dot(xn, wr_ref[...], preferred_element_type=jnp.float32)
        for h in range(H):
            xl_h = xl[:, h * O:(h + 1) * O]
            xlT_h = xlT[h * O:(h + 1) * O, :]
            att_h = att_ref[h, :].reshape(1, O, 1)

            def tile(i, _):
                d0 = i * DT
                xr_t = xr_s[pl.ds(d0, DT), h * O:(h + 1) * O].reshape(DT, O, 1)
                z = xlT_h[None, :, :] + xr_t
                z = jnp.where(z >= 0.0, z, 0.2 * z)
                s_t = jnp.sum(z * att_h, axis=1)
                m_t = cm_ref[0, pl.ds(d0, DT), :]
                rows = jax.lax.broadcasted_iota(jnp.int32, (DT, NP), 0) + d0
                cols = jax.lax.broadcasted_iota(jnp.int32, (DT, NP), 1)
                m_t = m_t + (rows == cols).astype(jnp.float32)
                ex = jnp.exp(s_t)
                den = jnp.sum(ex * m_t, axis=1, keepdims=True)
                alc = ex * m_t / (den + 1e-16)
                o_t = jnp.dot(alc, xl_h, preferred_element_type=jnp.float32,
                              precision=_HI)
                out_ref[0, pl.ds(d0, DT), h * O:(h + 1) * O] = (
                    o_t + b_ref[0][None, h * O:(h + 1) * O])
                return 0

            jax.lax.fori_loop(0, NP // DT, tile, 0)
        o = out_ref[0]
        ps_ref[0], pq_ref[0] = _colsums(o)
    return body


def _gat_layer(pre, stats, gambet, cm, Wl, Wr, att, b, H, O):
    cin = pre.shape[-1]
    cout = H * O
    in_specs = [pl.BlockSpec((1, NP, cin), lambda g: (g, 0, 0)),
                _full((G, 1, cin)), _full((G, 1, cin)),
                _full((1, cin)), _full((1, cin)),
                pl.BlockSpec((1, NP, NP), lambda g: (g, 0, 0)),
                _full((cin, cout)), _full((cin, cout)),
                _full((H, O)), _full((1, cout))]
    out, ps2, pq2 = pl.pallas_call(
        _gat_body(H, O, 32),
        grid=(G,),
        in_specs=in_specs,
        out_specs=[pl.BlockSpec((1, NP, cout), lambda g: (g, 0, 0)),
                   pl.BlockSpec((1, 1, cout), lambda g: (g, 0, 0)),
                   pl.BlockSpec((1, 1, cout), lambda g: (g, 0, 0))],
        out_shape=[_f32(G, NP, cout), _f32(G, 1, cout), _f32(G, 1, cout)],
        scratch_shapes=[pltpu.VMEM((NP, cout), jnp.float32)],
        interpret=_INTERPRET,
    )(pre, stats[0], stats[1], gambet[0], gambet[1], cm, Wl, Wr, att,
      b.reshape(1, cout))
    return out, (ps2, pq2)


_RB = 64
_BIG = 1 << 24


def _adj_build(ei, ea):
    mesh = plsc.VectorSubcoreMesh(core_axis_name="c", subcore_axis_name="s")

    @functools.partial(
        pl.kernel, mesh=mesh,
        compiler_params=pltpu.CompilerParams(needs_layout_passes=False),
        out_type=[jax.ShapeDtypeStruct((G, NP * NP), jnp.float32)] * 3,
        scratch_types=[
            pltpu.VMEM((EP,), jnp.int32),
            pltpu.VMEM((EP,), jnp.int32),
            pltpu.VMEM((EP,), jnp.float32),
            pltpu.VMEM((_RB * NP,), jnp.float32),
            pltpu.VMEM((_RB * NP,), jnp.float32),
            pltpu.VMEM((_RB * NP,), jnp.float32),
        ])
    def k(ei_hbm, ea_hbm, wacc_hbm, cnt_hbm, adj_hbm, s_v, d_v, w_v, wb, cb, ab):
        t = lax.axis_index("s") * 2 + lax.axis_index("c")
        pltpu.sync_copy(ei_hbm.at[0, pl.ds(t * EP, EP)], s_v)
        pltpu.sync_copy(ei_hbm.at[1, pl.ds(t * EP, EP)], d_v)
        pltpu.sync_copy(ea_hbm.at[pl.ds(t * EP, EP)], w_v)
        base = t * NP
        lane = lax.iota(jnp.int32, 16)
        z16 = jnp.zeros((16,), jnp.float32)
        one16 = jnp.ones((16,), jnp.float32)
        for r in range(NP // _RB):
            def zero(i, c):
                wb[pl.ds(i * 16, 16)] = z16
                cb[pl.ds(i * 16, 16)] = z16
                ab[pl.ds(i * 16, 16)] = z16
                return c
            lax.fori_loop(0, _RB * NP // 16, zero, 0)
            r0 = r * _RB

            def vec(v, c):
                sg = s_v[pl.ds(v * 16, 16)]
                dg = d_v[pl.ds(v * 16, 16)]
                w = w_v[pl.ds(v * 16, 16)]
                dl = dg - (base + r0)
                sl = sg - base
                valid = (dl >= 0) & (dl < _RB)
                idx = jnp.where(valid, dl * NP + sl, 0)
                plsc.addupdate_scatter(wb, [idx], w, mask=valid)
                plsc.addupdate_scatter(cb, [idx], one16, mask=valid)
                plsc.store_scatter(ab, [idx], w, mask=valid)
                return c
            lax.fori_loop(0, EP // 16, vec, 0)

            def fix(v, c):
                sg = s_v[pl.ds(v * 16, 16)]
                dg = d_v[pl.ds(v * 16, 16)]
                w = w_v[pl.ds(v * 16, 16)]
                dl = dg - (base + r0)
                sl = sg - base
                valid = (dl >= 0) & (dl < _RB)
                idx = jnp.where(valid, dl * NP + sl, 0)
                cnt = plsc.load_gather(cb, [idx], mask=valid)
                flg = valid & (cnt >= 2.0)
                nfl = jnp.max(plsc.all_reduce_population_count(flg))

                @pl.when(nfl > 0)
                def _():
                    for l in range(16):
                        plsc.store_scatter(ab, [idx], w, mask=flg & (lane == l))
                return c
            lax.fori_loop(0, EP // 16, fix, 0)
            pltpu.sync_copy(wb, wacc_hbm.at[t, pl.ds(r0 * NP, _RB * NP)])
            pltpu.sync_copy(cb, cnt_hbm.at[t, pl.ds(r0 * NP, _RB * NP)])
            pltpu.sync_copy(ab, adj_hbm.at[t, pl.ds(r0 * NP, _RB * NP)])

    wacc, cnt, adjt = k(ei, ea)
    return (wacc.reshape(G, NP, NP), cnt.reshape(G, NP, NP),
            adjt.reshape(G, NP, NP))


def _prep_body(wacc_ref, cm_ref, aw_ref, a1_ref):
    rows = jax.lax.broadcasted_iota(jnp.int32, (NP, NP), 0)
    cols = jax.lax.broadcasted_iota(jnp.int32, (NP, NP), 1)
    eye = (rows == cols).astype(jnp.float32)
    for src, dst in ((wacc_ref, aw_ref), (cm_ref, a1_ref)):
        m = src[0] + eye
        deg = jnp.sum(m, axis=1)
        dis = 1.0 / jnp.sqrt(deg)
        dst[0] = m * dis[:, None] * dis[None, :]


def _prep_adj(wacc, cm):
    spec = pl.BlockSpec((1, NP, NP), lambda g: (g, 0, 0))
    return pl.pallas_call(
        _prep_body,
        grid=(G,),
        in_specs=[spec, spec],
        out_specs=[spec, spec],
        out_shape=[_f32(G, NP, NP), _f32(G, NP, NP)],
        interpret=_INTERPRET,
    )(wacc, cm)


def _head_body(pre_ref, ps_in, pq_in, g_ref, be_ref, alls_ref, adj_ref,
               p2w_ref, p2b_ref, ow_ref, ob_ref, out_ref):
    xn = _bn_relu(pre_ref[0], ps_in[...], pq_in[...], g_ref[...], be_ref[...])
    s = alls_ref[0]
    s = jnp.exp(s - jnp.max(s, axis=-1, keepdims=True))
    s = s / jnp.sum(s, axis=-1, keepdims=True)
    dn = (((0,), (0,)), ((), ()))
    nodes = jax.lax.dot_general(s, xn, dn, preferred_element_type=jnp.float32)
    adjt = adj_ref[0]
    t1 = jax.lax.dot_general(adjt, s, dn, preferred_element_type=jnp.float32)
    oadj = jax.lax.dot_general(s, t1, dn, preferred_element_type=jnp.float32)
    eye5 = (jax.lax.broadcasted_iota(jnp.int32, (5, 5), 0)
            == jax.lax.broadcasted_iota(jnp.int32, (5, 5), 1)).astype(jnp.float32)
    a = oadj + eye5
    deg = jnp.clip(jnp.sum(a, axis=-1), 1.0, None)
    dis = 1.0 / jnp.sqrt(deg)
    an = a * dis[:, None] * dis[None, :]
    hw = jnp.dot(nodes, p2w_ref[...], preferred_element_type=jnp.float32)
    s2 = jnp.dot(an, hw, preferred_element_type=jnp.float32) + p2b_ref[0][None, :]
    s2 = jnp.exp(s2 - jnp.max(s2, axis=-1, keepdims=True))
    s2 = s2 / jnp.sum(s2, axis=-1, keepdims=True)
    xp = jax.lax.dot_general(s2, nodes, dn, preferred_element_type=jnp.float32)
    res = jnp.dot(xp, ow_ref[...], preferred_element_type=jnp.float32) + ob_ref[...]
    out_ref[0] = res


def _head(pre, stats, gambet, all_s, adj, p2w, p2b, ow, ob):
    in_specs = [pl.BlockSpec((1, NP, BSH), lambda g: (g, 0, 0)),
                _full((G, 1, BSH)), _full((G, 1, BSH)), _full((1, BSH)), _full((1, BSH)),
                pl.BlockSpec((1, NP, 5), lambda g: (g, 0, 0)),
                pl.BlockSpec((1, NP, NP), lambda g: (g, 0, 0)),
                _full((BSH, 1)), _full((1, 1)), _full((BSH, NOUT)), _full((1, NOUT))]
    return pl.pallas_call(
        _head_body,
        grid=(G,),
        in_specs=in_specs,
        out_specs=pl.BlockSpec((1, 1, NOUT), lambda g: (g, 0, 0)),
        out_shape=_f32(G, 1, NOUT),
        interpret=_INTERPRET,
    )(pre, stats[0], stats[1], gambet[0], gambet[1], all_s, adj,
      p2w, p2b.reshape(1, 1), ow, ob.reshape(1, NOUT))


def kernel(x, edge_index, edge_attr, params):
    P = params
    gb = lambda nm: (P[nm + '_bng'].reshape(1, -1), P[nm + '_bnb'].reshape(1, -1))

    wacc, cm, adjt = _adj_build(edge_index, edge_attr)
    aw, a1 = _prep_adj(wacc, cm)

    xg = x.reshape(G, NP, DIN)
    cur, st = _gcn_layer(xg, None, None, aw, P['enc0_W'], P['enc0_b'], first=True)
    saves = {}
    prev = 'enc0'
    for nm in ['enc1', 'enc2', 'enc3', 'enc4', 'enc5', 'enc6',
               'enc7', 'enc8', 'enc9', 'enc10', 'enc11']:
        cur, st = _gcn_layer(cur, st, gb(prev), aw, P[nm + '_W'], P[nm + '_b'])
        prev = nm
        if nm in ('enc2', 'enc5', 'enc8'):
            saves[nm] = (cur, st)
    cur, st = _gat_layer(cur, st, gb('enc11'), cm, P['attg0_Wl'], P['attg0_Wr'],
                         P['attg0_att'], P['attg0_b'], 2, 32)
    cur, st = _gcn2_layer(cur, st, gb('attg0'), *saves['enc8'], gb('enc8'), aw, P['attc2a_W'])
    cur, st = _gcn_layer(cur, st, gb('attc2a'), aw, P['attc0_W'], P['attc0_b'])
    cur, st = _gat_layer(cur, st, gb('attc0'), cm, P['attg1_Wl'], P['attg1_Wr'],
                         P['attg1_att'], P['attg1_b'], 2, 64)
    cur, st = _gcn2_layer(cur, st, gb('attg1'), *saves['enc5'], gb('enc5'), aw, P['decc2a_W'])
    cur, st = _gcn_layer(cur, st, gb('decc2a'), aw, P['dec0_W'], P['dec0_b'])
    cur, st = _gcn_layer(cur, st, gb('dec0'), a1, P['dec1_W'], P['dec1_b'])
    cur, st = _gcn2_layer(cur, st, gb('dec1'), *saves['enc2'], gb('enc2'), aw, P['decc2b_W'])
    cur, st = _gcn_layer(cur, st, gb('decc2b'), aw, P['dec2_W'], P['dec2_b'])
    cur, st = _gcn_layer(cur, st, gb('dec2'), aw, P['dec3_W'], P['dec3_b'])

    all_s, _ = _gcn_layer(cur, st, gb('dec3'), aw, P['pool1_W'], P['pool1_b'])
    return _head(cur, st, gb('dec3'), all_s, adjt,
                 P['pool2_W'], P['pool2_b'], P['out_W'], P['out_b']).reshape(G, NOUT)


# speedup vs baseline: 31.9202x; 1.0254x over previous
import functools
import math

import jax
import jax.numpy as jnp
from jax import lax
from jax.experimental import pallas as pl
from jax.experimental.pallas import tpu as pltpu
from jax.experimental.pallas import tpu_sc as plsc

G = 32
NP = 512
EP = NP * 16
N = G * NP
E = G * EP
DIN = 128
BSH = 256
NOUT = 10
ALPHA = 0.5
BETA = math.log(0.1 / 2.0 + 1.0)

_INTERPRET = False


def _dot3(a, b):
    f32, bf16 = jnp.float32, jnp.bfloat16
    ah = a.astype(bf16)
    al = (a - ah.astype(f32)).astype(bf16)
    bh = b.astype(bf16)
    bl = (b - bh.astype(f32)).astype(bf16)
    mm = lambda x, y: jax.lax.dot_general(
        x, y, (((1,), (0,)), ((), ())), preferred_element_type=f32)
    return mm(ah, bh) + (mm(ah, bl) + mm(al, bh))


def _f32(*shape):
    return jax.ShapeDtypeStruct(shape, jnp.float32)


def _bn_relu(pre, psum, psumsq, gam, bet):
    mean = jnp.sum(psum[:, 0, :], axis=0) / N
    msq = jnp.sum(psumsq[:, 0, :], axis=0) / N
    var = msq - mean * mean
    inv = gam[0] / jnp.sqrt(var + 1e-5)
    return jnp.maximum((pre - mean[None, :]) * inv[None, :] + bet[0][None, :], 0.0)


def _colsums(o):
    return jnp.sum(o, axis=0)[None, :], jnp.sum(o * o, axis=0)[None, :]


def _gcn_body(first):
    if first:
        def body(pre_ref, a_ref, w_ref, b_ref, out_ref, ps_ref, pq_ref):
            xn = pre_ref[0]
            h = jnp.dot(xn, w_ref[...], preferred_element_type=jnp.float32)
            o = _dot3(a_ref[0], h) + b_ref[0][None, :]
            out_ref[0] = o
            ps_ref[0], pq_ref[0] = _colsums(o)
        return body

    def body(pre_ref, ps_in, pq_in, g_ref, be_ref, a_ref, w_ref, b_ref,
             out_ref, ps_ref, pq_ref):
        xn = _bn_relu(pre_ref[0], ps_in[...], pq_in[...], g_ref[...], be_ref[...])
        h = jnp.dot(xn, w_ref[...], preferred_element_type=jnp.float32)
        o = _dot3(a_ref[0], h) + b_ref[0][None, :]
        out_ref[0] = o
        ps_ref[0], pq_ref[0] = _colsums(o)
    return body


def _full(shape):
    nd = len(shape)
    return pl.BlockSpec(shape, lambda g: (0,) * nd)


def _gcn_layer(pre, stats, gambet, A, W, b, first=False):
    cin = pre.shape[-1]
    cout = W.shape[-1]
    in_specs = [pl.BlockSpec((1, NP, cin), lambda g: (g, 0, 0))]
    args = [pre]
    if not first:
        ps, pq = stats
        gam, bet = gambet
        in_specs += [_full((G, 1, cin)), _full((G, 1, cin)),
                     _full((1, cin)), _full((1, cin))]
        args += [ps, pq, gam, bet]
    in_specs += [pl.BlockSpec((1, NP, NP), lambda g: (g, 0, 0)),
                 _full(W.shape), _full((1, cout))]
    args += [A, W, b.reshape(1, cout)]
    out, ps2, pq2 = pl.pallas_call(
        _gcn_body(first),
        grid=(G,),
        in_specs=in_specs,
        out_specs=[pl.BlockSpec((1, NP, cout), lambda g: (g, 0, 0)),
                   pl.BlockSpec((1, 1, cout), lambda g: (g, 0, 0)),
                   pl.BlockSpec((1, 1, cout), lambda g: (g, 0, 0))],
        out_shape=[_f32(G, NP, cout), _f32(G, 1, cout), _f32(G, 1, cout)],
        interpret=_INTERPRET,
    )(*args)
    return out, (ps2, pq2)


def _gcn2_body(pre_ref, ps_in, pq_in, g_ref, be_ref,
               p0_ref, ps0, pq0, g0_ref, be0_ref,
               a_ref, w_ref, out_ref, ps_ref, pq_ref):
    xn = _bn_relu(pre_ref[0], ps_in[...], pq_in[...], g_ref[...], be_ref[...])
    x0 = _bn_relu(p0_ref[0], ps0[...], pq0[...], g0_ref[...], be0_ref[...])
    h = _dot3(a_ref[0], xn)
    o = (1.0 - ALPHA) * h + ALPHA * x0
    o = (1.0 - BETA) * o + BETA * jnp.dot(o, w_ref[...], preferred_element_type=jnp.float32)
    out_ref[0] = o
    ps_ref[0], pq_ref[0] = _colsums(o)


def _gcn2_layer(pre, stats, gambet, pre0, stats0, gambet0, A, W):
    c = pre.shape[-1]
    in_specs = [pl.BlockSpec((1, NP, c), lambda g: (g, 0, 0)),
                _full((G, 1, c)), _full((G, 1, c)), _full((1, c)), _full((1, c)),
                pl.BlockSpec((1, NP, c), lambda g: (g, 0, 0)),
                _full((G, 1, c)), _full((G, 1, c)), _full((1, c)), _full((1, c)),
                pl.BlockSpec((1, NP, NP), lambda g: (g, 0, 0)),
                _full((c, c))]
    out, ps2, pq2 = pl.pallas_call(
        _gcn2_body,
        grid=(G,),
        in_specs=in_specs,
        out_specs=[pl.BlockSpec((1, NP, c), lambda g: (g, 0, 0)),
                   pl.BlockSpec((1, 1, c), lambda g: (g, 0, 0)),
                   pl.BlockSpec((1, 1, c), lambda g: (g, 0, 0))],
        out_shape=[_f32(G, NP, c), _f32(G, 1, c), _f32(G, 1, c)],
        interpret=_INTERPRET,
    )(pre, stats[0], stats[1], gambet[0], gambet[1],
      pre0, stats0[0], stats0[1], gambet0[0], gambet0[1], A, W)
    return out, (ps2, pq2)


def _gat_body(H, O, DT):
    def body(pre_ref, ps_in, pq_in, g_ref, be_ref, cm_ref,
             wl_ref, wr_ref, att_ref, b_ref, out_ref, ps_ref, pq_ref, xr_s):
        xn = _bn_relu(pre_ref[0], ps_in[...], pq_in[...], g_ref[...], be_ref[...])
        dnT = (((0,), (1,)), ((), ()))
        xl = jnp.dot(xn, wl_ref[...], preferred_element_type=jnp.float32)
        xlT = jax.lax.dot_general(wl_ref[...], xn, dnT,
                                  preferred_element_type=jnp.float32)
        xr_s[...] = jnp.dot(xn, wr_ref[...], preferred_element_type=jnp.float32)
        for h in range(H):
            xl_h = xl[:, h * O:(h + 1) * O]
            xlT_h = xlT[h * O:(h + 1) * O, :]
            att_h = att_ref[h, :].reshape(1, O, 1)

            def tile(i, _):
                d0 = i * DT
                xr_t = xr_s[pl.ds(d0, DT), h * O:(h + 1) * O].reshape(DT, O, 1)
                z = xlT_h[None, :, :] + xr_t
                z = jnp.where(z >= 0.0, z, 0.2 * z)
                s_t = jnp.sum(z * att_h, axis=1)
                m_t = cm_ref[0, pl.ds(d0, DT), :]
                rows = jax.lax.broadcasted_iota(jnp.int32, (DT, NP), 0) + d0
                cols = jax.lax.broadcasted_iota(jnp.int32, (DT, NP), 1)
                m_t = m_t + (rows == cols).astype(jnp.float32)
                ex = jnp.exp(s_t)
                den = jnp.sum(ex * m_t, axis=1, keepdims=True)
                alc = ex * m_t / (den + 1e-16)
                o_t = _dot3(alc, xl_h)
                out_ref[0, pl.ds(d0, DT), h * O:(h + 1) * O] = (
                    o_t + b_ref[0][None, h * O:(h + 1) * O])
                return 0

            jax.lax.fori_loop(0, NP // DT, tile, 0)
        o = out_ref[0]
        ps_ref[0], pq_ref[0] = _colsums(o)
    return body


def _gat_layer(pre, stats, gambet, cm, Wl, Wr, att, b, H, O):
    cin = pre.shape[-1]
    cout = H * O
    in_specs = [pl.BlockSpec((1, NP, cin), lambda g: (g, 0, 0)),
                _full((G, 1, cin)), _full((G, 1, cin)),
                _full((1, cin)), _full((1, cin)),
                pl.BlockSpec((1, NP, NP), lambda g: (g, 0, 0)),
                _full((cin, cout)), _full((cin, cout)),
                _full((H, O)), _full((1, cout))]
    out, ps2, pq2 = pl.pallas_call(
        _gat_body(H, O, 32),
        grid=(G,),
        in_specs=in_specs,
        out_specs=[pl.BlockSpec((1, NP, cout), lambda g: (g, 0, 0)),
                   pl.BlockSpec((1, 1, cout), lambda g: (g, 0, 0)),
                   pl.BlockSpec((1, 1, cout), lambda g: (g, 0, 0))],
        out_shape=[_f32(G, NP, cout), _f32(G, 1, cout), _f32(G, 1, cout)],
        scratch_shapes=[pltpu.VMEM((NP, cout), jnp.float32)],
        interpret=_INTERPRET,
    )(pre, stats[0], stats[1], gambet[0], gambet[1], cm, Wl, Wr, att,
      b.reshape(1, cout))
    return out, (ps2, pq2)


_RB = 64
_BIG = 1 << 24


def _adj_build(ei, ea):
    mesh = plsc.VectorSubcoreMesh(core_axis_name="c", subcore_axis_name="s")

    @functools.partial(
        pl.kernel, mesh=mesh,
        compiler_params=pltpu.CompilerParams(needs_layout_passes=False),
        out_type=[jax.ShapeDtypeStruct((G, NP * NP), jnp.float32)] * 3,
        scratch_types=[
            pltpu.VMEM((EP,), jnp.int32),
            pltpu.VMEM((EP,), jnp.int32),
            pltpu.VMEM((EP,), jnp.float32),
            pltpu.VMEM((_RB * NP,), jnp.float32),
            pltpu.VMEM((_RB * NP,), jnp.float32),
            pltpu.VMEM((_RB * NP,), jnp.float32),
        ])
    def k(ei_hbm, ea_hbm, wacc_hbm, cnt_hbm, adj_hbm, s_v, d_v, w_v, wb, cb, ab):
        t = lax.axis_index("s") * 2 + lax.axis_index("c")
        pltpu.sync_copy(ei_hbm.at[0, pl.ds(t * EP, EP)], s_v)
        pltpu.sync_copy(ei_hbm.at[1, pl.ds(t * EP, EP)], d_v)
        pltpu.sync_copy(ea_hbm.at[pl.ds(t * EP, EP)], w_v)
        base = t * NP
        lane = lax.iota(jnp.int32, 16)
        z16 = jnp.zeros((16,), jnp.float32)
        one16 = jnp.ones((16,), jnp.float32)
        for r in range(NP // _RB):
            def zero(i, c):
                wb[pl.ds(i * 16, 16)] = z16
                cb[pl.ds(i * 16, 16)] = z16
                ab[pl.ds(i * 16, 16)] = z16
                return c
            lax.fori_loop(0, _RB * NP // 16, zero, 0)
            r0 = r * _RB

            def vec(v, c):
                sg = s_v[pl.ds(v * 16, 16)]
                dg = d_v[pl.ds(v * 16, 16)]
                w = w_v[pl.ds(v * 16, 16)]
                dl = dg - (base + r0)
                sl = sg - base
                valid = (dl >= 0) & (dl < _RB)
                idx = jnp.where(valid, dl * NP + sl, 0)
                plsc.addupdate_scatter(wb, [idx], w, mask=valid)
                plsc.addupdate_scatter(cb, [idx], one16, mask=valid)
                plsc.store_scatter(ab, [idx], w, mask=valid)
                return c
            lax.fori_loop(0, EP // 16, vec, 0)

            def fix(v, c):
                sg = s_v[pl.ds(v * 16, 16)]
                dg = d_v[pl.ds(v * 16, 16)]
                w = w_v[pl.ds(v * 16, 16)]
                dl = dg - (base + r0)
                sl = sg - base
                valid = (dl >= 0) & (dl < _RB)
                idx = jnp.where(valid, dl * NP + sl, 0)
                cnt = plsc.load_gather(cb, [idx], mask=valid)
                flg = valid & (cnt >= 2.0)
                nfl = jnp.max(plsc.all_reduce_population_count(flg))

                @pl.when(nfl > 0)
                def _():
                    for l in range(16):
                        plsc.store_scatter(ab, [idx], w, mask=flg & (lane == l))
                return c
            lax.fori_loop(0, EP // 16, fix, 0)
            pltpu.sync_copy(wb, wacc_hbm.at[t, pl.ds(r0 * NP, _RB * NP)])
            pltpu.sync_copy(cb, cnt_hbm.at[t, pl.ds(r0 * NP, _RB * NP)])
            pltpu.sync_copy(ab, adj_hbm.at[t, pl.ds(r0 * NP, _RB * NP)])

    wacc, cnt, adjt = k(ei, ea)
    return (wacc.reshape(G, NP, NP), cnt.reshape(G, NP, NP),
            adjt.reshape(G, NP, NP))


def _prep_body(wacc_ref, cm_ref, aw_ref, a1_ref):
    rows = jax.lax.broadcasted_iota(jnp.int32, (NP, NP), 0)
    cols = jax.lax.broadcasted_iota(jnp.int32, (NP, NP), 1)
    eye = (rows == cols).astype(jnp.float32)
    for src, dst in ((wacc_ref, aw_ref), (cm_ref, a1_ref)):
        m = src[0] + eye
        deg = jnp.sum(m, axis=1)
        dis = 1.0 / jnp.sqrt(deg)
        dst[0] = m * dis[:, None] * dis[None, :]


def _prep_adj(wacc, cm):
    spec = pl.BlockSpec((1, NP, NP), lambda g: (g, 0, 0))
    return pl.pallas_call(
        _prep_body,
        grid=(G,),
        in_specs=[spec, spec],
        out_specs=[spec, spec],
        out_shape=[_f32(G, NP, NP), _f32(G, NP, NP)],
        interpret=_INTERPRET,
    )(wacc, cm)


def _head_body(pre_ref, ps_in, pq_in, g_ref, be_ref, alls_ref, adj_ref,
               p2w_ref, p2b_ref, ow_ref, ob_ref, out_ref):
    xn = _bn_relu(pre_ref[0], ps_in[...], pq_in[...], g_ref[...], be_ref[...])
    s = alls_ref[0]
    s = jnp.exp(s - jnp.max(s, axis=-1, keepdims=True))
    s = s / jnp.sum(s, axis=-1, keepdims=True)
    dn = (((0,), (0,)), ((), ()))
    nodes = jax.lax.dot_general(s, xn, dn, preferred_element_type=jnp.float32)
    adjt = adj_ref[0]
    t1 = jax.lax.dot_general(adjt, s, dn, preferred_element_type=jnp.float32)
    oadj = jax.lax.dot_general(s, t1, dn, preferred_element_type=jnp.float32)
    eye5 = (jax.lax.broadcasted_iota(jnp.int32, (5, 5), 0)
            == jax.lax.broadcasted_iota(jnp.int32, (5, 5), 1)).astype(jnp.float32)
    a = oadj + eye5
    deg = jnp.clip(jnp.sum(a, axis=-1), 1.0, None)
    dis = 1.0 / jnp.sqrt(deg)
    an = a * dis[:, None] * dis[None, :]
    hw = jnp.dot(nodes, p2w_ref[...], preferred_element_type=jnp.float32)
    s2 = jnp.dot(an, hw, preferred_element_type=jnp.float32) + p2b_ref[0][None, :]
    s2 = jnp.exp(s2 - jnp.max(s2, axis=-1, keepdims=True))
    s2 = s2 / jnp.sum(s2, axis=-1, keepdims=True)
    xp = jax.lax.dot_general(s2, nodes, dn, preferred_element_type=jnp.float32)
    res = jnp.dot(xp, ow_ref[...], preferred_element_type=jnp.float32) + ob_ref[...]
    out_ref[0] = res


def _head(pre, stats, gambet, all_s, adj, p2w, p2b, ow, ob):
    in_specs = [pl.BlockSpec((1, NP, BSH), lambda g: (g, 0, 0)),
                _full((G, 1, BSH)), _full((G, 1, BSH)), _full((1, BSH)), _full((1, BSH)),
                pl.BlockSpec((1, NP, 5), lambda g: (g, 0, 0)),
                pl.BlockSpec((1, NP, NP), lambda g: (g, 0, 0)),
                _full((BSH, 1)), _full((1, 1)), _full((BSH, NOUT)), _full((1, NOUT))]
    return pl.pallas_call(
        _head_body,
        grid=(G,),
        in_specs=in_specs,
        out_specs=pl.BlockSpec((1, 1, NOUT), lambda g: (g, 0, 0)),
        out_shape=_f32(G, 1, NOUT),
        interpret=_INTERPRET,
    )(pre, stats[0], stats[1], gambet[0], gambet[1], all_s, adj,
      p2w, p2b.reshape(1, 1), ow, ob.reshape(1, NOUT))


def kernel(x, edge_index, edge_attr, params):
    P = params
    gb = lambda nm: (P[nm + '_bng'].reshape(1, -1), P[nm + '_bnb'].reshape(1, -1))

    wacc, cm, adjt = _adj_build(edge_index, edge_attr)
    aw, a1 = _prep_adj(wacc, cm)

    xg = x.reshape(G, NP, DIN)
    cur, st = _gcn_layer(xg, None, None, aw, P['enc0_W'], P['enc0_b'], first=True)
    saves = {}
    prev = 'enc0'
    for nm in ['enc1', 'enc2', 'enc3', 'enc4', 'enc5', 'enc6',
               'enc7', 'enc8', 'enc9', 'enc10', 'enc11']:
        cur, st = _gcn_layer(cur, st, gb(prev), aw, P[nm + '_W'], P[nm + '_b'])
        prev = nm
        if nm in ('enc2', 'enc5', 'enc8'):
            saves[nm] = (cur, st)
    cur, st = _gat_layer(cur, st, gb('enc11'), cm, P['attg0_Wl'], P['attg0_Wr'],
                         P['attg0_att'], P['attg0_b'], 2, 32)
    cur, st = _gcn2_layer(cur, st, gb('attg0'), *saves['enc8'], gb('enc8'), aw, P['attc2a_W'])
    cur, st = _gcn_layer(cur, st, gb('attc2a'), aw, P['attc0_W'], P['attc0_b'])
    cur, st = _gat_layer(cur, st, gb('attc0'), cm, P['attg1_Wl'], P['attg1_Wr'],
                         P['attg1_att'], P['attg1_b'], 2, 64)
    cur, st = _gcn2_layer(cur, st, gb('attg1'), *saves['enc5'], gb('enc5'), aw, P['decc2a_W'])
    cur, st = _gcn_layer(cur, st, gb('decc2a'), aw, P['dec0_W'], P['dec0_b'])
    cur, st = _gcn_layer(cur, st, gb('dec0'), a1, P['dec1_W'], P['dec1_b'])
    cur, st = _gcn2_layer(cur, st, gb('dec1'), *saves['enc2'], gb('enc2'), aw, P['decc2b_W'])
    cur, st = _gcn_layer(cur, st, gb('decc2b'), aw, P['dec2_W'], P['dec2_b'])
    cur, st = _gcn_layer(cur, st, gb('dec2'), aw, P['dec3_W'], P['dec3_b'])

    all_s, _ = _gcn_layer(cur, st, gb('dec3'), aw, P['pool1_W'], P['pool1_b'])
    return _head(cur, st, gb('dec3'), all_s, adjt,
                 P['pool2_W'], P['pool2_b'], P['out_W'], P['out_b']).reshape(G, NOUT)


# speedup vs baseline: 36.3729x; 1.1395x over previous
import functools
import math

import jax
import jax.numpy as jnp
from jax import lax
from jax.experimental import pallas as pl
from jax.experimental.pallas import tpu as pltpu
from jax.experimental.pallas import tpu_sc as plsc

G = 32
NP = 512
EP = NP * 16
N = G * NP
E = G * EP
DIN = 128
BSH = 256
NOUT = 10
ALPHA = 0.5
BETA = math.log(0.1 / 2.0 + 1.0)

_INTERPRET = False


def _dot3(a, b):
    f32, bf16 = jnp.float32, jnp.bfloat16
    ah = a.astype(bf16)
    al = (a - ah.astype(f32)).astype(bf16)
    bh = b.astype(bf16)
    bl = (b - bh.astype(f32)).astype(bf16)
    mm = lambda x, y: jax.lax.dot_general(
        x, y, (((1,), (0,)), ((), ())), preferred_element_type=f32)
    return mm(ah, bh) + (mm(ah, bl) + mm(al, bh))


def _f32(*shape):
    return jax.ShapeDtypeStruct(shape, jnp.float32)


def _bn_relu(pre, psum, psumsq, gam, bet):
    mean = jnp.sum(psum[:, 0, :], axis=0) / N
    msq = jnp.sum(psumsq[:, 0, :], axis=0) / N
    var = msq - mean * mean
    inv = gam[0] / jnp.sqrt(var + 1e-5)
    return jnp.maximum((pre - mean[None, :]) * inv[None, :] + bet[0][None, :], 0.0)


def _colsums(o):
    return jnp.sum(o, axis=0)[None, :], jnp.sum(o * o, axis=0)[None, :]


def _gcn_body(first):
    if first:
        def body(pre_ref, a_ref, w_ref, b_ref, out_ref, ps_ref, pq_ref):
            xn = pre_ref[0]
            h = jnp.dot(xn, w_ref[...], preferred_element_type=jnp.float32)
            o = _dot3(a_ref[0], h) + b_ref[0][None, :]
            out_ref[0] = o
            ps_ref[0], pq_ref[0] = _colsums(o)
        return body

    def body(pre_ref, ps_in, pq_in, g_ref, be_ref, a_ref, w_ref, b_ref,
             out_ref, ps_ref, pq_ref):
        xn = _bn_relu(pre_ref[0], ps_in[...], pq_in[...], g_ref[...], be_ref[...])
        h = jnp.dot(xn, w_ref[...], preferred_element_type=jnp.float32)
        o = _dot3(a_ref[0], h) + b_ref[0][None, :]
        out_ref[0] = o
        ps_ref[0], pq_ref[0] = _colsums(o)
    return body


def _full(shape):
    nd = len(shape)
    return pl.BlockSpec(shape, lambda g: (0,) * nd)


def _gcn_layer(pre, stats, gambet, A, W, b, first=False):
    cin = pre.shape[-1]
    cout = W.shape[-1]
    in_specs = [pl.BlockSpec((1, NP, cin), lambda g: (g, 0, 0))]
    args = [pre]
    if not first:
        ps, pq = stats
        gam, bet = gambet
        in_specs += [_full((G, 1, cin)), _full((G, 1, cin)),
                     _full((1, cin)), _full((1, cin))]
        args += [ps, pq, gam, bet]
    in_specs += [pl.BlockSpec((1, NP, NP), lambda g: (g, 0, 0)),
                 _full(W.shape), _full((1, cout))]
    args += [A, W, b.reshape(1, cout)]
    out, ps2, pq2 = pl.pallas_call(
        _gcn_body(first),
        grid=(G,),
        in_specs=in_specs,
        out_specs=[pl.BlockSpec((1, NP, cout), lambda g: (g, 0, 0)),
                   pl.BlockSpec((1, 1, cout), lambda g: (g, 0, 0)),
                   pl.BlockSpec((1, 1, cout), lambda g: (g, 0, 0))],
        out_shape=[_f32(G, NP, cout), _f32(G, 1, cout), _f32(G, 1, cout)],
        interpret=_INTERPRET,
    )(*args)
    return out, (ps2, pq2)


def _gcn2_body(pre_ref, ps_in, pq_in, g_ref, be_ref,
               p0_ref, ps0, pq0, g0_ref, be0_ref,
               a_ref, w_ref, out_ref, ps_ref, pq_ref):
    xn = _bn_relu(pre_ref[0], ps_in[...], pq_in[...], g_ref[...], be_ref[...])
    x0 = _bn_relu(p0_ref[0], ps0[...], pq0[...], g0_ref[...], be0_ref[...])
    h = _dot3(a_ref[0], xn)
    o = (1.0 - ALPHA) * h + ALPHA * x0
    o = (1.0 - BETA) * o + BETA * jnp.dot(o, w_ref[...], preferred_element_type=jnp.float32)
    out_ref[0] = o
    ps_ref[0], pq_ref[0] = _colsums(o)


def _gcn2_layer(pre, stats, gambet, pre0, stats0, gambet0, A, W):
    c = pre.shape[-1]
    in_specs = [pl.BlockSpec((1, NP, c), lambda g: (g, 0, 0)),
                _full((G, 1, c)), _full((G, 1, c)), _full((1, c)), _full((1, c)),
                pl.BlockSpec((1, NP, c), lambda g: (g, 0, 0)),
                _full((G, 1, c)), _full((G, 1, c)), _full((1, c)), _full((1, c)),
                pl.BlockSpec((1, NP, NP), lambda g: (g, 0, 0)),
                _full((c, c))]
    out, ps2, pq2 = pl.pallas_call(
        _gcn2_body,
        grid=(G,),
        in_specs=in_specs,
        out_specs=[pl.BlockSpec((1, NP, c), lambda g: (g, 0, 0)),
                   pl.BlockSpec((1, 1, c), lambda g: (g, 0, 0)),
                   pl.BlockSpec((1, 1, c), lambda g: (g, 0, 0))],
        out_shape=[_f32(G, NP, c), _f32(G, 1, c), _f32(G, 1, c)],
        interpret=_INTERPRET,
    )(pre, stats[0], stats[1], gambet[0], gambet[1],
      pre0, stats0[0], stats0[1], gambet0[0], gambet0[1], A, W)
    return out, (ps2, pq2)


def _gat_body(H, O, DT):
    def body(pre_ref, ps_in, pq_in, g_ref, be_ref, cm_ref,
             wl_ref, wr_ref, att_ref, b_ref, out_ref, ps_ref, pq_ref, xr_s):
        xn = _bn_relu(pre_ref[0], ps_in[...], pq_in[...], g_ref[...], be_ref[...])
        dnT = (((0,), (1,)), ((), ()))
        xl = jnp.dot(xn, wl_ref[...], preferred_element_type=jnp.float32)
        xlT = jax.lax.dot_general(wl_ref[...], xn, dnT,
                                  preferred_element_type=jnp.float32)
        xr_s[...] = jnp.dot(xn, wr_ref[...], preferred_element_type=jnp.float32)
        for h in range(H):
            xl_h = xl[:, h * O:(h + 1) * O]
            xlT_h = xlT[h * O:(h + 1) * O, :]
            att_h = att_ref[h, :].reshape(1, O, 1)
            pT = _dot3(att_ref[h, :].reshape(1, O), xlT_h)

            def tile(i, _):
                d0 = i * DT
                xr_t = xr_s[pl.ds(d0, DT), h * O:(h + 1) * O].reshape(DT, O, 1)
                q_t = jnp.sum(xr_t * att_h, axis=1)
                zb = jnp.abs(xlT_h[None, :, :] + xr_t)
                s_t = 0.6 * (pT + q_t) + 0.4 * jnp.sum(zb * att_h, axis=1)
                m_t = cm_ref[0, pl.ds(d0, DT), :]
                rows = jax.lax.broadcasted_iota(jnp.int32, (DT, NP), 0) + d0
                cols = jax.lax.broadcasted_iota(jnp.int32, (DT, NP), 1)
                m_t = m_t + (rows == cols).astype(jnp.float32)
                ex = jnp.exp(s_t)
                den = jnp.sum(ex * m_t, axis=1, keepdims=True)
                alc = ex * m_t / (den + 1e-16)
                o_t = _dot3(alc, xl_h)
                out_ref[0, pl.ds(d0, DT), h * O:(h + 1) * O] = (
                    o_t + b_ref[0][None, h * O:(h + 1) * O])
                return 0

            jax.lax.fori_loop(0, NP // DT, tile, 0)
        o = out_ref[0]
        ps_ref[0], pq_ref[0] = _colsums(o)
    return body


def _gat_layer(pre, stats, gambet, cm, Wl, Wr, att, b, H, O):
    cin = pre.shape[-1]
    cout = H * O
    in_specs = [pl.BlockSpec((1, NP, cin), lambda g: (g, 0, 0)),
                _full((G, 1, cin)), _full((G, 1, cin)),
                _full((1, cin)), _full((1, cin)),
                pl.BlockSpec((1, NP, NP), lambda g: (g, 0, 0)),
                _full((cin, cout)), _full((cin, cout)),
                _full((H, O)), _full((1, cout))]
    out, ps2, pq2 = pl.pallas_call(
        _gat_body(H, O, 32),
        grid=(G,),
        in_specs=in_specs,
        out_specs=[pl.BlockSpec((1, NP, cout), lambda g: (g, 0, 0)),
                   pl.BlockSpec((1, 1, cout), lambda g: (g, 0, 0)),
                   pl.BlockSpec((1, 1, cout), lambda g: (g, 0, 0))],
        out_shape=[_f32(G, NP, cout), _f32(G, 1, cout), _f32(G, 1, cout)],
        scratch_shapes=[pltpu.VMEM((NP, cout), jnp.float32)],
        interpret=_INTERPRET,
    )(pre, stats[0], stats[1], gambet[0], gambet[1], cm, Wl, Wr, att,
      b.reshape(1, cout))
    return out, (ps2, pq2)


_RB = 64
_BIG = 1 << 24


def _adj_build(ei, ea):
    mesh = plsc.VectorSubcoreMesh(core_axis_name="c", subcore_axis_name="s")

    @functools.partial(
        pl.kernel, mesh=mesh,
        compiler_params=pltpu.CompilerParams(needs_layout_passes=False),
        out_type=[jax.ShapeDtypeStruct((G, NP * NP), jnp.float32)] * 3,
        scratch_types=[
            pltpu.VMEM((EP,), jnp.int32),
            pltpu.VMEM((EP,), jnp.int32),
            pltpu.VMEM((EP,), jnp.float32),
            pltpu.VMEM((_RB * NP,), jnp.float32),
            pltpu.VMEM((_RB * NP,), jnp.float32),
            pltpu.VMEM((_RB * NP,), jnp.float32),
        ])
    def k(ei_hbm, ea_hbm, wacc_hbm, cnt_hbm, adj_hbm, s_v, d_v, w_v, wb, cb, ab):
        t = lax.axis_index("s") * 2 + lax.axis_index("c")
        pltpu.sync_copy(ei_hbm.at[0, pl.ds(t * EP, EP)], s_v)
        pltpu.sync_copy(ei_hbm.at[1, pl.ds(t * EP, EP)], d_v)
        pltpu.sync_copy(ea_hbm.at[pl.ds(t * EP, EP)], w_v)
        base = t * NP
        lane = lax.iota(jnp.int32, 16)
        z16 = jnp.zeros((16,), jnp.float32)
        one16 = jnp.ones((16,), jnp.float32)
        for r in range(NP // _RB):
            def zero(i, c):
                wb[pl.ds(i * 16, 16)] = z16
                cb[pl.ds(i * 16, 16)] = z16
                ab[pl.ds(i * 16, 16)] = z16
                return c
            lax.fori_loop(0, _RB * NP // 16, zero, 0)
            r0 = r * _RB

            def vec(v, c):
                sg = s_v[pl.ds(v * 16, 16)]
                dg = d_v[pl.ds(v * 16, 16)]
                w = w_v[pl.ds(v * 16, 16)]
                dl = dg - (base + r0)
                sl = sg - base
                valid = (dl >= 0) & (dl < _RB)
                idx = jnp.where(valid, dl * NP + sl, 0)
                plsc.addupdate_scatter(wb, [idx], w, mask=valid)
                plsc.addupdate_scatter(cb, [idx], one16, mask=valid)
                plsc.store_scatter(ab, [idx], w, mask=valid)
                return c
            lax.fori_loop(0, EP // 16, vec, 0)

            def fix(v, c):
                sg = s_v[pl.ds(v * 16, 16)]
                dg = d_v[pl.ds(v * 16, 16)]
                w = w_v[pl.ds(v * 16, 16)]
                dl = dg - (base + r0)
                sl = sg - base
                valid = (dl >= 0) & (dl < _RB)
                idx = jnp.where(valid, dl * NP + sl, 0)
                cnt = plsc.load_gather(cb, [idx], mask=valid)
                flg = valid & (cnt >= 2.0)
                nfl = jnp.max(plsc.all_reduce_population_count(flg))

                @pl.when(nfl > 0)
                def _():
                    for l in range(16):
                        plsc.store_scatter(ab, [idx], w, mask=flg & (lane == l))
                return c
            lax.fori_loop(0, EP // 16, fix, 0)
            pltpu.sync_copy(wb, wacc_hbm.at[t, pl.ds(r0 * NP, _RB * NP)])
            pltpu.sync_copy(cb, cnt_hbm.at[t, pl.ds(r0 * NP, _RB * NP)])
            pltpu.sync_copy(ab, adj_hbm.at[t, pl.ds(r0 * NP, _RB * NP)])

    wacc, cnt, adjt = k(ei, ea)
    return (wacc.reshape(G, NP, NP), cnt.reshape(G, NP, NP),
            adjt.reshape(G, NP, NP))


def _prep_body(wacc_ref, cm_ref, aw_ref, a1_ref):
    rows = jax.lax.broadcasted_iota(jnp.int32, (NP, NP), 0)
    cols = jax.lax.broadcasted_iota(jnp.int32, (NP, NP), 1)
    eye = (rows == cols).astype(jnp.float32)
    for src, dst in ((wacc_ref, aw_ref), (cm_ref, a1_ref)):
        m = src[0] + eye
        deg = jnp.sum(m, axis=1)
        dis = 1.0 / jnp.sqrt(deg)
        dst[0] = m * dis[:, None] * dis[None, :]


def _prep_adj(wacc, cm):
    spec = pl.BlockSpec((1, NP, NP), lambda g: (g, 0, 0))
    return pl.pallas_call(
        _prep_body,
        grid=(G,),
        in_specs=[spec, spec],
        out_specs=[spec, spec],
        out_shape=[_f32(G, NP, NP), _f32(G, NP, NP)],
        interpret=_INTERPRET,
    )(wacc, cm)


def _head_body(pre_ref, ps_in, pq_in, g_ref, be_ref, alls_ref, adj_ref,
               p2w_ref, p2b_ref, ow_ref, ob_ref, out_ref):
    xn = _bn_relu(pre_ref[0], ps_in[...], pq_in[...], g_ref[...], be_ref[...])
    s = alls_ref[0]
    s = jnp.exp(s - jnp.max(s, axis=-1, keepdims=True))
    s = s / jnp.sum(s, axis=-1, keepdims=True)
    dn = (((0,), (0,)), ((), ()))
    nodes = jax.lax.dot_general(s, xn, dn, preferred_element_type=jnp.float32)
    adjt = adj_ref[0]
    t1 = jax.lax.dot_general(adjt, s, dn, preferred_element_type=jnp.float32)
    oadj = jax.lax.dot_general(s, t1, dn, preferred_element_type=jnp.float32)
    eye5 = (jax.lax.broadcasted_iota(jnp.int32, (5, 5), 0)
            == jax.lax.broadcasted_iota(jnp.int32, (5, 5), 1)).astype(jnp.float32)
    a = oadj + eye5
    deg = jnp.clip(jnp.sum(a, axis=-1), 1.0, None)
    dis = 1.0 / jnp.sqrt(deg)
    an = a * dis[:, None] * dis[None, :]
    hw = jnp.dot(nodes, p2w_ref[...], preferred_element_type=jnp.float32)
    s2 = jnp.dot(an, hw, preferred_element_type=jnp.float32) + p2b_ref[0][None, :]
    s2 = jnp.exp(s2 - jnp.max(s2, axis=-1, keepdims=True))
    s2 = s2 / jnp.sum(s2, axis=-1, keepdims=True)
    xp = jax.lax.dot_general(s2, nodes, dn, preferred_element_type=jnp.float32)
    res = jnp.dot(xp, ow_ref[...], preferred_element_type=jnp.float32) + ob_ref[...]
    out_ref[0] = res


def _head(pre, stats, gambet, all_s, adj, p2w, p2b, ow, ob):
    in_specs = [pl.BlockSpec((1, NP, BSH), lambda g: (g, 0, 0)),
                _full((G, 1, BSH)), _full((G, 1, BSH)), _full((1, BSH)), _full((1, BSH)),
                pl.BlockSpec((1, NP, 5), lambda g: (g, 0, 0)),
                pl.BlockSpec((1, NP, NP), lambda g: (g, 0, 0)),
                _full((BSH, 1)), _full((1, 1)), _full((BSH, NOUT)), _full((1, NOUT))]
    return pl.pallas_call(
        _head_body,
        grid=(G,),
        in_specs=in_specs,
        out_specs=pl.BlockSpec((1, 1, NOUT), lambda g: (g, 0, 0)),
        out_shape=_f32(G, 1, NOUT),
        interpret=_INTERPRET,
    )(pre, stats[0], stats[1], gambet[0], gambet[1], all_s, adj,
      p2w, p2b.reshape(1, 1), ow, ob.reshape(1, NOUT))


def kernel(x, edge_index, edge_attr, params):
    P = params
    gb = lambda nm: (P[nm + '_bng'].reshape(1, -1), P[nm + '_bnb'].reshape(1, -1))

    wacc, cm, adjt = _adj_build(edge_index, edge_attr)
    aw, a1 = _prep_adj(wacc, cm)

    xg = x.reshape(G, NP, DIN)
    cur, st = _gcn_layer(xg, None, None, aw, P['enc0_W'], P['enc0_b'], first=True)
    saves = {}
    prev = 'enc0'
    for nm in ['enc1', 'enc2', 'enc3', 'enc4', 'enc5', 'enc6',
               'enc7', 'enc8', 'enc9', 'enc10', 'enc11']:
        cur, st = _gcn_layer(cur, st, gb(prev), aw, P[nm + '_W'], P[nm + '_b'])
        prev = nm
        if nm in ('enc2', 'enc5', 'enc8'):
            saves[nm] = (cur, st)
    cur, st = _gat_layer(cur, st, gb('enc11'), cm, P['attg0_Wl'], P['attg0_Wr'],
                         P['attg0_att'], P['attg0_b'], 2, 32)
    cur, st = _gcn2_layer(cur, st, gb('attg0'), *saves['enc8'], gb('enc8'), aw, P['attc2a_W'])
    cur, st = _gcn_layer(cur, st, gb('attc2a'), aw, P['attc0_W'], P['attc0_b'])
    cur, st = _gat_layer(cur, st, gb('attc0'), cm, P['attg1_Wl'], P['attg1_Wr'],
                         P['attg1_att'], P['attg1_b'], 2, 64)
    cur, st = _gcn2_layer(cur, st, gb('attg1'), *saves['enc5'], gb('enc5'), aw, P['decc2a_W'])
    cur, st = _gcn_layer(cur, st, gb('decc2a'), aw, P['dec0_W'], P['dec0_b'])
    cur, st = _gcn_layer(cur, st, gb('dec0'), a1, P['dec1_W'], P['dec1_b'])
    cur, st = _gcn2_layer(cur, st, gb('dec1'), *saves['enc2'], gb('enc2'), aw, P['decc2b_W'])
    cur, st = _gcn_layer(cur, st, gb('decc2b'), aw, P['dec2_W'], P['dec2_b'])
    cur, st = _gcn_layer(cur, st, gb('dec2'), aw, P['dec3_W'], P['dec3_b'])

    all_s, _ = _gcn_layer(cur, st, gb('dec3'), aw, P['pool1_W'], P['pool1_b'])
    return _head(cur, st, gb('dec3'), all_s, adjt,
                 P['pool2_W'], P['pool2_b'], P['out_W'], P['out_b']).reshape(G, NOUT)


# speedup vs baseline: 36.4232x; 1.0014x over previous
import functools
import math

import jax
import jax.numpy as jnp
from jax import lax
from jax.experimental import pallas as pl
from jax.experimental.pallas import tpu as pltpu
from jax.experimental.pallas import tpu_sc as plsc

G = 32
NP = 512
EP = NP * 16
N = G * NP
E = G * EP
DIN = 128
BSH = 256
NOUT = 10
ALPHA = 0.5
BETA = math.log(0.1 / 2.0 + 1.0)


def _dot3(a, b):
    f32, bf16 = jnp.float32, jnp.bfloat16
    ah = a.astype(bf16)
    al = (a - ah.astype(f32)).astype(bf16)
    bh = b.astype(bf16)
    bl = (b - bh.astype(f32)).astype(bf16)
    mm = lambda x, y: jax.lax.dot_general(
        x, y, (((1,), (0,)), ((), ())), preferred_element_type=f32)
    return mm(ah, bh) + (mm(ah, bl) + mm(al, bh))


def _f32(*shape):
    return jax.ShapeDtypeStruct(shape, jnp.float32)


def _bn_relu(pre, psum, psumsq, gam, bet):
    mean = jnp.sum(psum[:, 0, :], axis=0) / N
    msq = jnp.sum(psumsq[:, 0, :], axis=0) / N
    var = msq - mean * mean
    inv = gam[0] / jnp.sqrt(var + 1e-5)
    return jnp.maximum((pre - mean[None, :]) * inv[None, :] + bet[0][None, :], 0.0)


def _colsums(o):
    return jnp.sum(o, axis=0)[None, :], jnp.sum(o * o, axis=0)[None, :]


def _gcn_body(first):
    if first:
        def body(pre_ref, a_ref, w_ref, b_ref, out_ref, ps_ref, pq_ref):
            xn = pre_ref[0]
            h = jnp.dot(xn, w_ref[...], preferred_element_type=jnp.float32)
            o = _dot3(a_ref[0], h) + b_ref[0][None, :]
            out_ref[0] = o
            ps_ref[0], pq_ref[0] = _colsums(o)
        return body

    def body(pre_ref, ps_in, pq_in, g_ref, be_ref, a_ref, w_ref, b_ref,
             out_ref, ps_ref, pq_ref):
        xn = _bn_relu(pre_ref[0], ps_in[...], pq_in[...], g_ref[...], be_ref[...])
        h = jnp.dot(xn, w_ref[...], preferred_element_type=jnp.float32)
        o = _dot3(a_ref[0], h) + b_ref[0][None, :]
        out_ref[0] = o
        ps_ref[0], pq_ref[0] = _colsums(o)
    return body


def _full(shape):
    nd = len(shape)
    return pl.BlockSpec(shape, lambda g: (0,) * nd)


def _gcn_layer(pre, stats, gambet, A, W, b, first=False):
    cin = pre.shape[-1]
    cout = W.shape[-1]
    in_specs = [pl.BlockSpec((1, NP, cin), lambda g: (g, 0, 0))]
    args = [pre]
    if not first:
        ps, pq = stats
        gam, bet = gambet
        in_specs += [_full((G, 1, cin)), _full((G, 1, cin)),
                     _full((1, cin)), _full((1, cin))]
        args += [ps, pq, gam, bet]
    in_specs += [pl.BlockSpec((1, NP, NP), lambda g: (g, 0, 0)),
                 _full(W.shape), _full((1, cout))]
    args += [A, W, b.reshape(1, cout)]
    out, ps2, pq2 = pl.pallas_call(
        _gcn_body(first),
        grid=(G,),
        in_specs=in_specs,
        out_specs=[pl.BlockSpec((1, NP, cout), lambda g: (g, 0, 0)),
                   pl.BlockSpec((1, 1, cout), lambda g: (g, 0, 0)),
                   pl.BlockSpec((1, 1, cout), lambda g: (g, 0, 0))],
        out_shape=[_f32(G, NP, cout), _f32(G, 1, cout), _f32(G, 1, cout)],
    )(*args)
    return out, (ps2, pq2)


def _gcn2_body(pre_ref, ps_in, pq_in, g_ref, be_ref,
               p0_ref, ps0, pq0, g0_ref, be0_ref,
               a_ref, w_ref, out_ref, ps_ref, pq_ref):
    xn = _bn_relu(pre_ref[0], ps_in[...], pq_in[...], g_ref[...], be_ref[...])
    x0 = _bn_relu(p0_ref[0], ps0[...], pq0[...], g0_ref[...], be0_ref[...])
    h = _dot3(a_ref[0], xn)
    o = (1.0 - ALPHA) * h + ALPHA * x0
    o = (1.0 - BETA) * o + BETA * jnp.dot(o, w_ref[...], preferred_element_type=jnp.float32)
    out_ref[0] = o
    ps_ref[0], pq_ref[0] = _colsums(o)


def _gcn2_layer(pre, stats, gambet, pre0, stats0, gambet0, A, W):
    c = pre.shape[-1]
    in_specs = [pl.BlockSpec((1, NP, c), lambda g: (g, 0, 0)),
                _full((G, 1, c)), _full((G, 1, c)), _full((1, c)), _full((1, c)),
                pl.BlockSpec((1, NP, c), lambda g: (g, 0, 0)),
                _full((G, 1, c)), _full((G, 1, c)), _full((1, c)), _full((1, c)),
                pl.BlockSpec((1, NP, NP), lambda g: (g, 0, 0)),
                _full((c, c))]
    out, ps2, pq2 = pl.pallas_call(
        _gcn2_body,
        grid=(G,),
        in_specs=in_specs,
        out_specs=[pl.BlockSpec((1, NP, c), lambda g: (g, 0, 0)),
                   pl.BlockSpec((1, 1, c), lambda g: (g, 0, 0)),
                   pl.BlockSpec((1, 1, c), lambda g: (g, 0, 0))],
        out_shape=[_f32(G, NP, c), _f32(G, 1, c), _f32(G, 1, c)],
    )(pre, stats[0], stats[1], gambet[0], gambet[1],
      pre0, stats0[0], stats0[1], gambet0[0], gambet0[1], A, W)
    return out, (ps2, pq2)


def _gat_body(H, O, DT):
    def body(pre_ref, ps_in, pq_in, g_ref, be_ref, cm_ref,
             wl_ref, wr_ref, att_ref, b_ref, out_ref, ps_ref, pq_ref, xr_s):
        xn = _bn_relu(pre_ref[0], ps_in[...], pq_in[...], g_ref[...], be_ref[...])
        dnT = (((0,), (1,)), ((), ()))
        xl = jnp.dot(xn, wl_ref[...], preferred_element_type=jnp.float32)
        xlT = jax.lax.dot_general(wl_ref[...], xn, dnT,
                                  preferred_element_type=jnp.float32)
        xr_s[...] = jnp.dot(xn, wr_ref[...], preferred_element_type=jnp.float32)
        for h in range(H):
            xl_h = xl[:, h * O:(h + 1) * O]
            xlT_h = xlT[h * O:(h + 1) * O, :]
            att_h = att_ref[h, :].reshape(1, O, 1)
            pT = _dot3(att_ref[h, :].reshape(1, O), xlT_h)

            def tile(i, _):
                d0 = i * DT
                xr_t = xr_s[pl.ds(d0, DT), h * O:(h + 1) * O].reshape(DT, O, 1)
                q_t = jnp.sum(xr_t * att_h, axis=1)
                zb = jnp.abs(xlT_h[None, :, :] + xr_t)
                s_t = 0.6 * (pT + q_t) + 0.4 * jnp.sum(zb * att_h, axis=1)
                m_t = cm_ref[0, pl.ds(d0, DT), :]
                rows = jax.lax.broadcasted_iota(jnp.int32, (DT, NP), 0) + d0
                cols = jax.lax.broadcasted_iota(jnp.int32, (DT, NP), 1)
                m_t = m_t + (rows == cols).astype(jnp.float32)
                ex = jnp.exp(s_t)
                den = jnp.sum(ex * m_t, axis=1, keepdims=True)
                alc = ex * m_t / (den + 1e-16)
                o_t = _dot3(alc, xl_h)
                out_ref[0, pl.ds(d0, DT), h * O:(h + 1) * O] = (
                    o_t + b_ref[0][None, h * O:(h + 1) * O])
                return 0

            jax.lax.fori_loop(0, NP // DT, tile, 0)
        o = out_ref[0]
        ps_ref[0], pq_ref[0] = _colsums(o)
    return body


def _gat_layer(pre, stats, gambet, cm, Wl, Wr, att, b, H, O):
    cin = pre.shape[-1]
    cout = H * O
    in_specs = [pl.BlockSpec((1, NP, cin), lambda g: (g, 0, 0)),
                _full((G, 1, cin)), _full((G, 1, cin)),
                _full((1, cin)), _full((1, cin)),
                pl.BlockSpec((1, NP, NP), lambda g: (g, 0, 0)),
                _full((cin, cout)), _full((cin, cout)),
                _full((H, O)), _full((1, cout))]
    out, ps2, pq2 = pl.pallas_call(
        _gat_body(H, O, 32),
        grid=(G,),
        in_specs=in_specs,
        out_specs=[pl.BlockSpec((1, NP, cout), lambda g: (g, 0, 0)),
                   pl.BlockSpec((1, 1, cout), lambda g: (g, 0, 0)),
                   pl.BlockSpec((1, 1, cout), lambda g: (g, 0, 0))],
        out_shape=[_f32(G, NP, cout), _f32(G, 1, cout), _f32(G, 1, cout)],
        scratch_shapes=[pltpu.VMEM((NP, cout), jnp.float32)],
    )(pre, stats[0], stats[1], gambet[0], gambet[1], cm, Wl, Wr, att,
      b.reshape(1, cout))
    return out, (ps2, pq2)


_RB = 64
_BIG = 1 << 24


def _adj_build(ei, ea):
    mesh = plsc.VectorSubcoreMesh(core_axis_name="c", subcore_axis_name="s")

    @functools.partial(
        pl.kernel, mesh=mesh,
        compiler_params=pltpu.CompilerParams(needs_layout_passes=False),
        out_type=[jax.ShapeDtypeStruct((G, NP * NP), jnp.float32)] * 3,
        scratch_types=[
            pltpu.VMEM((EP,), jnp.int32),
            pltpu.VMEM((EP,), jnp.int32),
            pltpu.VMEM((EP,), jnp.float32),
            pltpu.VMEM((_RB * NP,), jnp.float32),
            pltpu.VMEM((_RB * NP,), jnp.float32),
            pltpu.VMEM((_RB * NP,), jnp.float32),
        ])
    def k(ei_hbm, ea_hbm, wacc_hbm, cnt_hbm, adj_hbm, s_v, d_v, w_v, wb, cb, ab):
        t = lax.axis_index("s") * 2 + lax.axis_index("c")
        pltpu.sync_copy(ei_hbm.at[0, pl.ds(t * EP, EP)], s_v)
        pltpu.sync_copy(ei_hbm.at[1, pl.ds(t * EP, EP)], d_v)
        pltpu.sync_copy(ea_hbm.at[pl.ds(t * EP, EP)], w_v)
        base = t * NP
        lane = lax.iota(jnp.int32, 16)
        z16 = jnp.zeros((16,), jnp.float32)
        one16 = jnp.ones((16,), jnp.float32)
        for r in range(NP // _RB):
            def zero(i, c):
                wb[pl.ds(i * 16, 16)] = z16
                cb[pl.ds(i * 16, 16)] = z16
                ab[pl.ds(i * 16, 16)] = z16
                return c
            lax.fori_loop(0, _RB * NP // 16, zero, 0)
            r0 = r * _RB

            def vec(v, c):
                sg = s_v[pl.ds(v * 16, 16)]
                dg = d_v[pl.ds(v * 16, 16)]
                w = w_v[pl.ds(v * 16, 16)]
                dl = dg - (base + r0)
                sl = sg - base
                valid = (dl >= 0) & (dl < _RB)
                idx = jnp.where(valid, dl * NP + sl, 0)
                plsc.addupdate_scatter(wb, [idx], w, mask=valid)
                plsc.addupdate_scatter(cb, [idx], one16, mask=valid)
                plsc.store_scatter(ab, [idx], w, mask=valid)
                return c
            lax.fori_loop(0, EP // 16, vec, 0)

            def fix(v, c):
                sg = s_v[pl.ds(v * 16, 16)]
                dg = d_v[pl.ds(v * 16, 16)]
                w = w_v[pl.ds(v * 16, 16)]
                dl = dg - (base + r0)
                sl = sg - base
                valid = (dl >= 0) & (dl < _RB)
                idx = jnp.where(valid, dl * NP + sl, 0)
                cnt = plsc.load_gather(cb, [idx], mask=valid)
                flg = valid & (cnt >= 2.0)
                nfl = jnp.max(plsc.all_reduce_population_count(flg))

                @pl.when(nfl > 0)
                def _():
                    for l in range(16):
                        plsc.store_scatter(ab, [idx], w, mask=flg & (lane == l))
                return c
            lax.fori_loop(0, EP // 16, fix, 0)
            pltpu.sync_copy(wb, wacc_hbm.at[t, pl.ds(r0 * NP, _RB * NP)])
            pltpu.sync_copy(cb, cnt_hbm.at[t, pl.ds(r0 * NP, _RB * NP)])
            pltpu.sync_copy(ab, adj_hbm.at[t, pl.ds(r0 * NP, _RB * NP)])

    wacc, cnt, adjt = k(ei, ea)
    return (wacc.reshape(G, NP, NP), cnt.reshape(G, NP, NP),
            adjt.reshape(G, NP, NP))


def _prep_body(wacc_ref, cm_ref, aw_ref, a1_ref):
    rows = jax.lax.broadcasted_iota(jnp.int32, (NP, NP), 0)
    cols = jax.lax.broadcasted_iota(jnp.int32, (NP, NP), 1)
    eye = (rows == cols).astype(jnp.float32)
    for src, dst in ((wacc_ref, aw_ref), (cm_ref, a1_ref)):
        m = src[0] + eye
        deg = jnp.sum(m, axis=1)
        dis = 1.0 / jnp.sqrt(deg)
        dst[0] = m * dis[:, None] * dis[None, :]


def _prep_adj(wacc, cm):
    spec = pl.BlockSpec((1, NP, NP), lambda g: (g, 0, 0))
    return pl.pallas_call(
        _prep_body,
        grid=(G,),
        in_specs=[spec, spec],
        out_specs=[spec, spec],
        out_shape=[_f32(G, NP, NP), _f32(G, NP, NP)],
    )(wacc, cm)


def _head_body(pre_ref, ps_in, pq_in, g_ref, be_ref, alls_ref, adj_ref,
               p2w_ref, p2b_ref, ow_ref, ob_ref, out_ref):
    xn = _bn_relu(pre_ref[0], ps_in[...], pq_in[...], g_ref[...], be_ref[...])
    s = alls_ref[0]
    s = jnp.exp(s - jnp.max(s, axis=-1, keepdims=True))
    s = s / jnp.sum(s, axis=-1, keepdims=True)
    dn = (((0,), (0,)), ((), ()))
    nodes = jax.lax.dot_general(s, xn, dn, preferred_element_type=jnp.float32)
    adjt = adj_ref[0]
    t1 = jax.lax.dot_general(adjt, s, dn, preferred_element_type=jnp.float32)
    oadj = jax.lax.dot_general(s, t1, dn, preferred_element_type=jnp.float32)
    eye5 = (jax.lax.broadcasted_iota(jnp.int32, (5, 5), 0)
            == jax.lax.broadcasted_iota(jnp.int32, (5, 5), 1)).astype(jnp.float32)
    a = oadj + eye5
    deg = jnp.clip(jnp.sum(a, axis=-1), 1.0, None)
    dis = 1.0 / jnp.sqrt(deg)
    an = a * dis[:, None] * dis[None, :]
    hw = jnp.dot(nodes, p2w_ref[...], preferred_element_type=jnp.float32)
    s2 = jnp.dot(an, hw, preferred_element_type=jnp.float32) + p2b_ref[0][None, :]
    s2 = jnp.exp(s2 - jnp.max(s2, axis=-1, keepdims=True))
    s2 = s2 / jnp.sum(s2, axis=-1, keepdims=True)
    xp = jax.lax.dot_general(s2, nodes, dn, preferred_element_type=jnp.float32)
    res = jnp.dot(xp, ow_ref[...], preferred_element_type=jnp.float32) + ob_ref[...]
    out_ref[0] = res


def _head(pre, stats, gambet, all_s, adj, p2w, p2b, ow, ob):
    in_specs = [pl.BlockSpec((1, NP, BSH), lambda g: (g, 0, 0)),
                _full((G, 1, BSH)), _full((G, 1, BSH)), _full((1, BSH)), _full((1, BSH)),
                pl.BlockSpec((1, NP, 5), lambda g: (g, 0, 0)),
                pl.BlockSpec((1, NP, NP), lambda g: (g, 0, 0)),
                _full((BSH, 1)), _full((1, 1)), _full((BSH, NOUT)), _full((1, NOUT))]
    return pl.pallas_call(
        _head_body,
        grid=(G,),
        in_specs=in_specs,
        out_specs=pl.BlockSpec((1, 1, NOUT), lambda g: (g, 0, 0)),
        out_shape=_f32(G, 1, NOUT),
    )(pre, stats[0], stats[1], gambet[0], gambet[1], all_s, adj,
      p2w, p2b.reshape(1, 1), ow, ob.reshape(1, NOUT))


def kernel(x, edge_index, edge_attr, params):
    P = params
    gb = lambda nm: (P[nm + '_bng'].reshape(1, -1), P[nm + '_bnb'].reshape(1, -1))

    wacc, cm, adjt = _adj_build(edge_index, edge_attr)
    aw, a1 = _prep_adj(wacc, cm)

    xg = x.reshape(G, NP, DIN)
    cur, st = _gcn_layer(xg, None, None, aw, P['enc0_W'], P['enc0_b'], first=True)
    saves = {}
    prev = 'enc0'
    for nm in ['enc1', 'enc2', 'enc3', 'enc4', 'enc5', 'enc6',
               'enc7', 'enc8', 'enc9', 'enc10', 'enc11']:
        cur, st = _gcn_layer(cur, st, gb(prev), aw, P[nm + '_W'], P[nm + '_b'])
        prev = nm
        if nm in ('enc2', 'enc5', 'enc8'):
            saves[nm] = (cur, st)
    cur, st = _gat_layer(cur, st, gb('enc11'), cm, P['attg0_Wl'], P['attg0_Wr'],
                         P['attg0_att'], P['attg0_b'], 2, 32)
    cur, st = _gcn2_layer(cur, st, gb('attg0'), *saves['enc8'], gb('enc8'), aw, P['attc2a_W'])
    cur, st = _gcn_layer(cur, st, gb('attc2a'), aw, P['attc0_W'], P['attc0_b'])
    cur, st = _gat_layer(cur, st, gb('attc0'), cm, P['attg1_Wl'], P['attg1_Wr'],
                         P['attg1_att'], P['attg1_b'], 2, 64)
    cur, st = _gcn2_layer(cur, st, gb('attg1'), *saves['enc5'], gb('enc5'), aw, P['decc2a_W'])
    cur, st = _gcn_layer(cur, st, gb('decc2a'), aw, P['dec0_W'], P['dec0_b'])
    cur, st = _gcn_layer(cur, st, gb('dec0'), a1, P['dec1_W'], P['dec1_b'])
    cur, st = _gcn2_layer(cur, st, gb('dec1'), *saves['enc2'], gb('enc2'), aw, P['decc2b_W'])
    cur, st = _gcn_layer(cur, st, gb('decc2b'), aw, P['dec2_W'], P['dec2_b'])
    cur, st = _gcn_layer(cur, st, gb('dec2'), aw, P['dec3_W'], P['dec3_b'])

    all_s, _ = _gcn_layer(cur, st, gb('dec3'), aw, P['pool1_W'], P['pool1_b'])
    return _head(cur, st, gb('dec3'), all_s, adjt,
                 P['pool2_W'], P['pool2_b'], P['out_W'], P['out_b']).reshape(G, NOUT)
